# baseline (speedup 1.0000x reference)
import functools

import jax
import jax.numpy as jnp
from jax import lax
from jax.experimental import pallas as pl
from jax.experimental.pallas import tpu as pltpu
from jax.experimental.pallas import tpu_sc as plsc

_N = 10000
_E = 320000
_H = 128
_HH = 64
_F32 = jnp.float32


_ET = 3200
_EG = _E // _ET
_NT = 2000
_NG = _N // _NT
_AGGW = 80

_row_spec_e = pl.BlockSpec((_ET, _H), lambda i: (i, 0))
_row_spec_n = pl.BlockSpec((_NT, _H), lambda i: (i, 0))
_w_spec = pl.BlockSpec((_H, _H), lambda i: (0, 0))
_b_spec = pl.BlockSpec((1, _H), lambda i: (0, 0))
_half_spec_e = pl.BlockSpec((2, _ET, _HH), lambda i: (0, i, 0))
_half_spec_n = pl.BlockSpec((2, _NT, _HH), lambda i: (0, i, 0))
_aggp_spec = pl.BlockSpec((2, _NT, _AGGW), lambda i: (0, i, 0))


def _ln_relu(v, g, b):
    mu = jnp.mean(v, axis=-1, keepdims=True)
    d = v - mu
    var = jnp.mean(d * d, axis=-1, keepdims=True)
    return jnp.maximum(d * lax.rsqrt(var + 1e-5) * g + b, 0.0)


def _dot(a, b):
    return jnp.dot(a, b, preferred_element_type=_F32)


def _split(out_ref, v):
    out_ref[0] = v[:, :_HH]
    out_ref[1] = v[:, _HH:]


def _join(ref):
    v = ref[...]
    return jnp.concatenate([v[0], v[1]], axis=-1)


def _node0_body(x_r, wne_r, bne_r, w0x_r, b0x_r, g_r, bt_r,
                wma_r, bm_r, wea_r, bea_r, web_r,
                h_r, u_r, a_r, b_r):
    h = _dot(x_r[...], wne_r[...]) + bne_r[...]
    h = _dot(h, w0x_r[...]) + b0x_r[...]
    h_r[...] = h
    hn = _ln_relu(h, g_r[...], bt_r[...])
    _split(u_r, _dot(hn, wma_r[...]) + bm_r[...])
    _split(a_r, _dot(hn, wea_r[...]) + bea_r[...])
    _split(b_r, _dot(hn, web_r[...]))


def _tc_node0(x, wne, bne, w0x, b0x, g, bt, wma, bm, wea, bea, web):
    tshp = jax.ShapeDtypeStruct((2, _N, _HH), _F32)
    return pl.pallas_call(
        _node0_body,
        grid=(_NG,),
        in_specs=[_row_spec_n, _w_spec, _b_spec, _w_spec, _b_spec, _b_spec,
                  _b_spec, _w_spec, _b_spec, _w_spec, _b_spec, _w_spec],
        out_specs=[_row_spec_n, _half_spec_n, _half_spec_n, _half_spec_n],
        out_shape=[jax.ShapeDtypeStruct((_N, _H), _F32), tshp, tshp, tshp],
    )(x, wne, bne, w0x, b0x, g, bt, wma, bm, wea, bea, web)


def _apply_agg(h_prev, aggp):
    agg = jnp.concatenate([aggp[0, :, :_HH], aggp[1, :, :_HH]], axis=-1)
    cnt = aggp[0, :, _HH:_HH + 1]
    return h_prev + agg / jnp.maximum(cnt, 1.0)


def _node_mid_body(h_r, aggp_r, g_r, bt_r, wma_r, bm_r, wea_r, bea_r, web_r,
                   hn_out_r, u_r, a_r, b_r):
    h = _apply_agg(h_r[...], aggp_r[...])
    hn_out_r[...] = h
    hn = _ln_relu(h, g_r[...], bt_r[...])
    _split(u_r, _dot(hn, wma_r[...]) + bm_r[...])
    _split(a_r, _dot(hn, wea_r[...]) + bea_r[...])
    _split(b_r, _dot(hn, web_r[...]))


def _tc_node_mid(h, aggp, g, bt, wma, bm, wea, bea, web):
    tshp = jax.ShapeDtypeStruct((2, _N, _HH), _F32)
    return pl.pallas_call(
        _node_mid_body,
        grid=(_NG,),
        in_specs=[_row_spec_n, _aggp_spec, _b_spec, _b_spec, _w_spec, _b_spec,
                  _w_spec, _b_spec, _w_spec],
        out_specs=[_row_spec_n, _half_spec_n, _half_spec_n, _half_spec_n],
        out_shape=[jax.ShapeDtypeStruct((_N, _H), _F32), tshp, tshp, tshp],
    )(h, aggp, g, bt, wma, bm, wea, bea, web)


def _node_fin_body(h_r, aggp_r, g_r, bt_r, wdab_r, pq_r):
    h = _apply_agg(h_r[...], aggp_r[...])
    hf = _ln_relu(h, g_r[...], bt_r[...])
    pq_r[...] = _dot(hf, wdab_r[...])


def _tc_node_fin(h, aggp, g, bt, wdab):
    return pl.pallas_call(
        _node_fin_body,
        grid=(_NG,),
        in_specs=[_row_spec_n, _aggp_spec, _b_spec, _b_spec,
                  pl.BlockSpec((_H, 2), lambda i: (0, 0))],
        out_specs=pl.BlockSpec((_NT, 2), lambda i: (i, 0)),
        out_shape=jax.ShapeDtypeStruct((_N, 2), _F32),
    )(h, aggp, g, bt, wdab)


def _edge0_body(ea_r, wee_r, bee_r, w0e_r, b0e_r, g_r, bt_r, wmb_r, wec_r,
                e_r, pm_r, pe_r):
    e = _dot(ea_r[...], wee_r[...]) + bee_r[...]
    e = _dot(e, w0e_r[...]) + b0e_r[...]
    _split(e_r, e)
    gn = _ln_relu(e, g_r[...], bt_r[...])
    _split(pm_r, _dot(gn, wmb_r[...]))
    _split(pe_r, _dot(gn, wec_r[...]))


def _tc_edge0(ea, wee, bee, w0e, b0e, g, bt, wmb, wec):
    shp = jax.ShapeDtypeStruct((2, _E, _HH), _F32)
    return pl.pallas_call(
        _edge0_body,
        grid=(_EG,),
        in_specs=[pl.BlockSpec((_ET, 16), lambda i: (i, 0)),
                  pl.BlockSpec((16, _H), lambda i: (0, 0)),
                  _b_spec, _w_spec, _b_spec, _b_spec, _b_spec, _w_spec, _w_spec],
        out_specs=[_half_spec_e] * 3,
        out_shape=[shp] * 3,
    )(ea, wee, bee, w0e, b0e, g, bt, wmb, wec)


def _edge_mid_body(e_r, g_r, bt_r, wmb_r, wec_r, pm_r, pe_r):
    gn = _ln_relu(_join(e_r), g_r[...], bt_r[...])
    _split(pm_r, _dot(gn, wmb_r[...]))
    _split(pe_r, _dot(gn, wec_r[...]))


def _tc_edge_mid(e, g, bt, wmb, wec):
    shp = jax.ShapeDtypeStruct((2, _E, _HH), _F32)
    return pl.pallas_call(
        _edge_mid_body,
        grid=(_EG,),
        in_specs=[_half_spec_e, _b_spec, _b_spec, _w_spec, _w_spec],
        out_specs=[_half_spec_e] * 2,
        out_shape=[shp] * 2,
    )(e, g, bt, wmb, wec)


def _edge_fin_body(e_r, g_r, bt_r, wdc_r, bd_r, r_r):
    ef = _ln_relu(_join(e_r), g_r[...], bt_r[...])
    r_r[...] = _dot(ef, wdc_r[...]) + bd_r[...]


def _tc_edge_fin(e, g, bt, wdc, bd):
    return pl.pallas_call(
        _edge_fin_body,
        grid=(_EG,),
        in_specs=[_half_spec_e, _b_spec, _b_spec,
                  pl.BlockSpec((_H, 1), lambda i: (0, 0)),
                  pl.BlockSpec((1, 1), lambda i: (0, 0))],
        out_specs=pl.BlockSpec((_ET, 1), lambda i: (i, 0)),
        out_shape=jax.ShapeDtypeStruct((_E, 1), _F32),
    )(e, g, bt, wdc, bd)


_EPS = _E // 16
_C = 80
_NCHUNK = _EPS // _C
_RPT = _N // 16
_ZR = 125
_QH = _HH // 16

_sc_mesh = plsc.VectorSubcoreMesh(core_axis_name="c", subcore_axis_name="s")
_sc_params = pltpu.CompilerParams(use_tc_tiling_on_sc=False,
                                  needs_layout_passes=False)


def _sc_layer_body(src_hbm, dst_hbm, e_hbm, pm_hbm, pe_hbm, u_hbm, a_hbm,
                   b_hbm, e_out, agg_out,
                   sidx, didx, gidx, ubuf, abuf, bbuf, ebuf, pmbuf, pebuf,
                   mbuf, zrow, acc, sem):
    c = lax.axis_index("c")
    s = lax.axis_index("s")
    ebase = s * _EPS
    hbase = c * (_E * _HH)

    zv = jnp.zeros((16,), _F32)

    def _zfill(j, _):
        for k in range(_AGGW // 16):
            zrow[j, pl.ds(k * 16, 16)] = zv
        return 0

    lax.fori_loop(0, _ZR, _zfill, 0)

    def _zcopy(t, _):
        pltpu.sync_copy(zrow, acc.at[pl.ds(s * _RPT + t * _ZR, _ZR)])
        return 0

    lax.fori_loop(0, _RPT // _ZR, _zcopy, 0)
    plsc.subcore_barrier()

    padv = jnp.where(lax.iota(jnp.int32, 16) == 0,
                     jnp.full((16,), 1.0, _F32), jnp.zeros((16,), _F32))

    def _padfill(j, _):
        mbuf[j, pl.ds(_HH, 16)] = padv
        return 0

    lax.fori_loop(0, _C, _padfill, 0)

    coff = c * _N

    def _chunk(i, _):
        eoff = ebase + i * _C
        pltpu.sync_copy(src_hbm.at[pl.ds(eoff, _C)], sidx)
        pltpu.sync_copy(dst_hbm.at[pl.ds(eoff, _C)], didx)
        foff = hbase + eoff * _HH
        pltpu.sync_copy(e_hbm.at[pl.ds(foff, _C * _HH)], ebuf)
        pltpu.sync_copy(pm_hbm.at[pl.ds(foff, _C * _HH)], pmbuf)
        pltpu.sync_copy(pe_hbm.at[pl.ds(foff, _C * _HH)], pebuf)

        def _shift(j, _):
            sl = pl.ds(j * 16, 16)
            gidx[sl] = sidx[sl] + coff
            return 0

        lax.fori_loop(0, _C // 16, _shift, 0)
        g1 = pltpu.async_copy(u_hbm.at[gidx], ubuf, sem)
        g2 = pltpu.async_copy(a_hbm.at[gidx], abuf, sem)
        g1.wait()
        g2.wait()

        def _shiftd(j, _):
            sl = pl.ds(j * 16, 16)
            gidx[sl] = didx[sl] + coff
            return 0

        lax.fori_loop(0, _C // 16, _shiftd, 0)
        pltpu.async_copy(b_hbm.at[gidx], bbuf, sem).wait()

        def _row(j, _):
            for k in range(_QH):
                sl = pl.ds(j * _HH + k * 16, 16)
                msl = pl.ds(k * 16, 16)
                mbuf[j, msl] = jnp.maximum(ubuf[j, msl] + pmbuf[sl], 0.0)
                ebuf[sl] = ebuf[sl] + jnp.maximum(
                    abuf[j, msl] + bbuf[j, msl] + pebuf[sl], 0.0)
            return 0

        lax.fori_loop(0, _C, _row, 0)
        pltpu.sync_copy(mbuf, acc.at[didx], add=True)
        pltpu.sync_copy(ebuf, e_out.at[pl.ds(foff, _C * _HH)])
        return 0

    lax.fori_loop(0, _NCHUNK, _chunk, 0)
    plsc.subcore_barrier()

    def _flush(t, _):
        r0 = s * _RPT + t * _ZR
        pltpu.sync_copy(acc.at[pl.ds(r0, _ZR)], agg_out.at[c, pl.ds(r0, _ZR)])
        return 0

    lax.fori_loop(0, _RPT // _ZR, _flush, 0)


_sc_layer = functools.partial(
    pl.kernel,
    mesh=_sc_mesh,
    compiler_params=_sc_params,
    out_type=[jax.ShapeDtypeStruct((2 * _E * _HH,), _F32),
              jax.ShapeDtypeStruct((2, _N, _AGGW), _F32)],
    scratch_types=[
        pltpu.VMEM((_C,), jnp.int32),
        pltpu.VMEM((_C,), jnp.int32),
        pltpu.VMEM((_C,), jnp.int32),
        pltpu.VMEM((_C, _HH), _F32),
        pltpu.VMEM((_C, _HH), _F32),
        pltpu.VMEM((_C, _HH), _F32),
        pltpu.VMEM((_C * _HH,), _F32),
        pltpu.VMEM((_C * _HH,), _F32),
        pltpu.VMEM((_C * _HH,), _F32),
        pltpu.VMEM((_C, _AGGW), _F32),
        pltpu.VMEM((_ZR, _AGGW), _F32),
        pltpu.VMEM_SHARED((_N, _AGGW), _F32),
        pltpu.SemaphoreType.DMA,
    ],
)(_sc_layer_body)


_EPW = _E // 32
_C2 = 2000


def _sc_dec_body(src_hbm, dst_hbm, r_hbm, p_hbm, q_hbm, z_out,
                 ptab, qtab, sbuf, dbuf, rbuf, zbuf):
    c = lax.axis_index("c")
    s = lax.axis_index("s")
    w = c * 16 + s
    base = w * _EPW
    pltpu.sync_copy(p_hbm, ptab)
    pltpu.sync_copy(q_hbm, qtab)

    def _chunk(i, _):
        off = base + i * _C2
        pltpu.sync_copy(src_hbm.at[pl.ds(off, _C2)], sbuf)
        pltpu.sync_copy(dst_hbm.at[pl.ds(off, _C2)], dbuf)
        pltpu.sync_copy(r_hbm.at[pl.ds(off, _C2)], rbuf)

        def _vec(t, _):
            sl = pl.ds(t * 16, 16)
            pg = plsc.load_gather(ptab, [sbuf[sl]])
            qg = plsc.load_gather(qtab, [dbuf[sl]])
            zbuf[sl] = jnp.maximum(pg + qg + rbuf[sl], 0.0)
            return 0

        lax.fori_loop(0, _C2 // 16, _vec, 0)
        pltpu.sync_copy(zbuf, z_out.at[pl.ds(off, _C2)])
        return 0

    lax.fori_loop(0, _EPW // _C2, _chunk, 0)


_sc_dec = functools.partial(
    pl.kernel,
    mesh=_sc_mesh,
    compiler_params=_sc_params,
    out_type=jax.ShapeDtypeStruct((_E,), _F32),
    scratch_types=[
        pltpu.VMEM((_N,), _F32),
        pltpu.VMEM((_N,), _F32),
        pltpu.VMEM((_C2,), jnp.int32),
        pltpu.VMEM((_C2,), jnp.int32),
        pltpu.VMEM((_C2,), _F32),
        pltpu.VMEM((_C2,), _F32),
    ],
)(_sc_dec_body)


def kernel(x, edge_index, edge_attr, W_ne, b_ne, W_ee, b_ee, W0x, b0x, W0e,
           b0e, gamma, beta, Wm, bm, We, be, W_dec, b_dec, scale):
    src = edge_index[0]
    dst = edge_index[1]
    r1 = lambda v: v.reshape(1, _H)
    flat = lambda v: v.reshape(2 * _E * _HH)
    tab = lambda v: v.reshape(2 * _N, _HH)
    g = [r1(gamma[0]), r1(gamma[1])]
    bt = [r1(beta[0]), r1(beta[1])]
    wma = [Wm[0, :_H], Wm[1, :_H]]
    wmb = [Wm[0, _H:], Wm[1, _H:]]
    wea = [We[0, :_H], We[1, :_H]]
    web = [We[0, _H:2 * _H], We[1, _H:2 * _H]]
    wec = [We[0, 2 * _H:], We[1, 2 * _H:]]
    bmr = [r1(bm[0]), r1(bm[1])]
    ber = [r1(be[0]), r1(be[1])]
    wdab = W_dec[:2 * _H].reshape(2, _H).T
    wdc = W_dec[2 * _H:]
    bd = b_dec.reshape(1, 1)

    h0, u0, a0, b0 = _tc_node0(x, W_ne, r1(b_ne), W0x, r1(b0x), g[0], bt[0],
                               wma[0], bmr[0], wea[0], ber[0], web[0])
    e0, pm0, pe0 = _tc_edge0(edge_attr, W_ee, r1(b_ee), W0e, r1(b0e), g[0],
                             bt[0], wmb[0], wec[0])
    e1f, aggp0 = _sc_layer(src, dst, flat(e0), flat(pm0), flat(pe0),
                           tab(u0), tab(a0), tab(b0))
    e1 = e1f.reshape(2, _E, _HH)
    h1, u1, a1, b1 = _tc_node_mid(h0, aggp0, g[1], bt[1], wma[1], bmr[1],
                                  wea[1], ber[1], web[1])
    pm1, pe1 = _tc_edge_mid(e1, g[1], bt[1], wmb[1], wec[1])
    e2f, aggp1 = _sc_layer(src, dst, e1f, flat(pm1), flat(pe1),
                           tab(u1), tab(a1), tab(b1))
    e2 = e2f.reshape(2, _E, _HH)
    pq = _tc_node_fin(h1, aggp1, g[0], bt[0], wdab)
    r = _tc_edge_fin(e2, g[0], bt[0], wdc, bd)
    z = _sc_dec(src, dst, r.reshape(_E), pq[:, 0], pq[:, 1])
    return (z * scale).reshape(_E, 1)


# speedup vs baseline: 1.5331x; 1.5331x over previous
import functools

import jax
import jax.numpy as jnp
from jax import lax
from jax.experimental import pallas as pl
from jax.experimental.pallas import tpu as pltpu
from jax.experimental.pallas import tpu_sc as plsc

_N = 10000
_E = 320000
_H = 128
_HH = 64
_F32 = jnp.float32


_ET = 3200
_EG = _E // _ET
_NT = 2000
_NG = _N // _NT
_AGGW = 80

_row_spec_e = pl.BlockSpec((_ET, _H), lambda i: (i, 0))
_row_spec_n = pl.BlockSpec((_NT, _H), lambda i: (i, 0))
_w_spec = pl.BlockSpec((_H, _H), lambda i: (0, 0))
_b_spec = pl.BlockSpec((1, _H), lambda i: (0, 0))
_half_spec_e = pl.BlockSpec((2, _ET, _HH), lambda i: (0, i, 0))
_half_spec_n = pl.BlockSpec((2, _NT, _HH), lambda i: (0, i, 0))
_aggp_spec = pl.BlockSpec((2, _NT, _AGGW), lambda i: (0, i, 0))


def _ln_relu(v, g, b):
    mu = jnp.mean(v, axis=-1, keepdims=True)
    d = v - mu
    var = jnp.mean(d * d, axis=-1, keepdims=True)
    return jnp.maximum(d * lax.rsqrt(var + 1e-5) * g + b, 0.0)


def _dot(a, b):
    return jnp.dot(a, b, preferred_element_type=_F32)


def _split(out_ref, v):
    out_ref[0] = v[:, :_HH]
    out_ref[1] = v[:, _HH:]


def _join(ref):
    v = ref[...]
    return jnp.concatenate([v[0], v[1]], axis=-1)


def _node0_body(x_r, wne_r, bne_r, w0x_r, b0x_r, g_r, bt_r,
                wma_r, bm_r, wea_r, bea_r, web_r,
                h_r, u_r, a_r, b_r):
    h = _dot(x_r[...], wne_r[...]) + bne_r[...]
    h = _dot(h, w0x_r[...]) + b0x_r[...]
    h_r[...] = h
    hn = _ln_relu(h, g_r[...], bt_r[...])
    _split(u_r, _dot(hn, wma_r[...]) + bm_r[...])
    _split(a_r, _dot(hn, wea_r[...]) + bea_r[...])
    _split(b_r, _dot(hn, web_r[...]))


def _tc_node0(x, wne, bne, w0x, b0x, g, bt, wma, bm, wea, bea, web):
    tshp = jax.ShapeDtypeStruct((2, _N, _HH), _F32)
    return pl.pallas_call(
        _node0_body,
        grid=(_NG,),
        in_specs=[_row_spec_n, _w_spec, _b_spec, _w_spec, _b_spec, _b_spec,
                  _b_spec, _w_spec, _b_spec, _w_spec, _b_spec, _w_spec],
        out_specs=[_row_spec_n, _half_spec_n, _half_spec_n, _half_spec_n],
        out_shape=[jax.ShapeDtypeStruct((_N, _H), _F32), tshp, tshp, tshp],
    )(x, wne, bne, w0x, b0x, g, bt, wma, bm, wea, bea, web)


def _apply_agg(h_prev, aggp):
    agg = jnp.concatenate([aggp[0, :, :_HH], aggp[1, :, :_HH]], axis=-1)
    cnt = aggp[0, :, _HH:_HH + 1]
    return h_prev + agg / jnp.maximum(cnt, 1.0)


def _node_mid_body(h_r, aggp_r, g_r, bt_r, wma_r, bm_r, wea_r, bea_r, web_r,
                   hn_out_r, u_r, a_r, b_r):
    h = _apply_agg(h_r[...], aggp_r[...])
    hn_out_r[...] = h
    hn = _ln_relu(h, g_r[...], bt_r[...])
    _split(u_r, _dot(hn, wma_r[...]) + bm_r[...])
    _split(a_r, _dot(hn, wea_r[...]) + bea_r[...])
    _split(b_r, _dot(hn, web_r[...]))


def _tc_node_mid(h, aggp, g, bt, wma, bm, wea, bea, web):
    tshp = jax.ShapeDtypeStruct((2, _N, _HH), _F32)
    return pl.pallas_call(
        _node_mid_body,
        grid=(_NG,),
        in_specs=[_row_spec_n, _aggp_spec, _b_spec, _b_spec, _w_spec, _b_spec,
                  _w_spec, _b_spec, _w_spec],
        out_specs=[_row_spec_n, _half_spec_n, _half_spec_n, _half_spec_n],
        out_shape=[jax.ShapeDtypeStruct((_N, _H), _F32), tshp, tshp, tshp],
    )(h, aggp, g, bt, wma, bm, wea, bea, web)


def _node_fin_body(h_r, aggp_r, g_r, bt_r, wdab_r, pq_r):
    h = _apply_agg(h_r[...], aggp_r[...])
    hf = _ln_relu(h, g_r[...], bt_r[...])
    pq_r[...] = _dot(hf, wdab_r[...])


def _tc_node_fin(h, aggp, g, bt, wdab):
    return pl.pallas_call(
        _node_fin_body,
        grid=(_NG,),
        in_specs=[_row_spec_n, _aggp_spec, _b_spec, _b_spec,
                  pl.BlockSpec((_H, 2), lambda i: (0, 0))],
        out_specs=pl.BlockSpec((_NT, 2), lambda i: (i, 0)),
        out_shape=jax.ShapeDtypeStruct((_N, 2), _F32),
    )(h, aggp, g, bt, wdab)


def _edge0_body(ea_r, wee_r, bee_r, w0e_r, b0e_r, g_r, bt_r, wmb_r, wec_r,
                e_r, pm_r, pe_r):
    e = _dot(ea_r[...], wee_r[...]) + bee_r[...]
    e = _dot(e, w0e_r[...]) + b0e_r[...]
    _split(e_r, e)
    gn = _ln_relu(e, g_r[...], bt_r[...])
    _split(pm_r, _dot(gn, wmb_r[...]))
    _split(pe_r, _dot(gn, wec_r[...]))


def _tc_edge0(ea, wee, bee, w0e, b0e, g, bt, wmb, wec):
    shp = jax.ShapeDtypeStruct((2, _E, _HH), _F32)
    return pl.pallas_call(
        _edge0_body,
        grid=(_EG,),
        in_specs=[pl.BlockSpec((_ET, 16), lambda i: (i, 0)),
                  pl.BlockSpec((16, _H), lambda i: (0, 0)),
                  _b_spec, _w_spec, _b_spec, _b_spec, _b_spec, _w_spec, _w_spec],
        out_specs=[_half_spec_e] * 3,
        out_shape=[shp] * 3,
    )(ea, wee, bee, w0e, b0e, g, bt, wmb, wec)


def _edge_mid_body(e_r, g_r, bt_r, wmb_r, wec_r, pm_r, pe_r):
    gn = _ln_relu(_join(e_r), g_r[...], bt_r[...])
    _split(pm_r, _dot(gn, wmb_r[...]))
    _split(pe_r, _dot(gn, wec_r[...]))


def _tc_edge_mid(e, g, bt, wmb, wec):
    shp = jax.ShapeDtypeStruct((2, _E, _HH), _F32)
    return pl.pallas_call(
        _edge_mid_body,
        grid=(_EG,),
        in_specs=[_half_spec_e, _b_spec, _b_spec, _w_spec, _w_spec],
        out_specs=[_half_spec_e] * 2,
        out_shape=[shp] * 2,
    )(e, g, bt, wmb, wec)


def _edge_fin_body(e_r, g_r, bt_r, wdc_r, bd_r, r_r):
    ef = _ln_relu(_join(e_r), g_r[...], bt_r[...])
    r_r[...] = _dot(ef, wdc_r[...]) + bd_r[...]


def _tc_edge_fin(e, g, bt, wdc, bd):
    return pl.pallas_call(
        _edge_fin_body,
        grid=(_EG,),
        in_specs=[_half_spec_e, _b_spec, _b_spec,
                  pl.BlockSpec((_H, 1), lambda i: (0, 0)),
                  pl.BlockSpec((1, 1), lambda i: (0, 0))],
        out_specs=pl.BlockSpec((_ET, 1), lambda i: (i, 0)),
        out_shape=jax.ShapeDtypeStruct((_E, 1), _F32),
    )(e, g, bt, wdc, bd)


_EPS = _E // 16
_C = 80
_NCHUNK = _EPS // _C
_RPT = _N // 16
_ZR = 25
_QH = _HH // 16

_sc_mesh = plsc.VectorSubcoreMesh(core_axis_name="c", subcore_axis_name="s")
_sc_params = pltpu.CompilerParams(use_tc_tiling_on_sc=False,
                                  needs_layout_passes=False)


_CH = _C * _HH
_NPAIR = _NCHUNK // 2


def _sc_layer_body(src_hbm, dst_hbm, e_hbm, pm_hbm, pe_hbm, u_hbm, a_hbm,
                   b_hbm, e_out, agg_out,
                   sidx, didx, gsx, gdx, dcp, ubuf, abuf, bbuf, ebuf, pmbuf,
                   pebuf, obuf, mbuf, zrow, acc, sem_i, sem_l, sem_g, sem_w):
    c = lax.axis_index("c")
    s = lax.axis_index("s")
    ebase = s * _EPS
    hbase = c * (_E * _HH)
    coff = c * _N

    zv = jnp.zeros((16,), _F32)

    def _zfill(j, _):
        for k in range(_AGGW // 16):
            zrow[j, pl.ds(k * 16, 16)] = zv
        return 0

    lax.fori_loop(0, _ZR, _zfill, 0)

    def _zcopy(t, _):
        pltpu.sync_copy(zrow, acc.at[pl.ds(s * _RPT + t * _ZR, _ZR)])
        return 0

    lax.fori_loop(0, _RPT // _ZR, _zcopy, 0)
    plsc.subcore_barrier()

    padv = jnp.where(lax.iota(jnp.int32, 16) == 0,
                     jnp.full((16,), 1.0, _F32), jnp.zeros((16,), _F32))

    def _padfill(j, _):
        mbuf[j, pl.ds(_HH, 16)] = padv
        return 0

    lax.fori_loop(0, _C, _padfill, 0)


    def _fire_idx(i):
        q = lax.rem(i, 4)
        eoff = ebase + i * _C
        pltpu.async_copy(src_hbm.at[pl.ds(eoff, _C)], sidx.at[q], sem_i)
        pltpu.async_copy(dst_hbm.at[pl.ds(eoff, _C)], didx.at[q], sem_i)

    def _fire_streams(b, i):
        foff = hbase + (ebase + i * _C) * _HH
        pltpu.async_copy(e_hbm.at[pl.ds(foff, _CH)],
                         ebuf.at[pl.ds(b * _CH, _CH)], sem_l)
        pltpu.async_copy(pm_hbm.at[pl.ds(foff, _CH)],
                         pmbuf.at[pl.ds(b * _CH, _CH)], sem_l)
        pltpu.async_copy(pe_hbm.at[pl.ds(foff, _CH)],
                         pebuf.at[pl.ds(b * _CH, _CH)], sem_l)

    def _arm_gathers(b, i):
        q = lax.rem(i, 4)
        pltpu.make_async_copy(src_hbm.at[pl.ds(0, _C)], sidx.at[0],
                              sem_i).wait()
        pltpu.make_async_copy(src_hbm.at[pl.ds(0, _C)], didx.at[0],
                              sem_i).wait()

        def _shift(j, _):
            sl = pl.ds(j * 16, 16)
            gsx[b, sl] = sidx[q, sl] + coff
            gdx[b, sl] = didx[q, sl] + coff
            dcp[b, sl] = didx[q, sl]
            return 0

        lax.fori_loop(0, _C // 16, _shift, 0)
        pltpu.async_copy(u_hbm.at[gsx.at[b]], ubuf.at[pl.ds(b * _C, _C)],
                         sem_g)
        pltpu.async_copy(a_hbm.at[gsx.at[b]], abuf.at[pl.ds(b * _C, _C)],
                         sem_g)
        pltpu.async_copy(b_hbm.at[gdx.at[b]], bbuf.at[pl.ds(b * _C, _C)],
                         sem_g)

    def _drain_streams(b):
        for buf in (ebuf, pmbuf, pebuf):
            pltpu.make_async_copy(e_hbm.at[pl.ds(0, _CH)],
                                  buf.at[pl.ds(b * _CH, _CH)], sem_l).wait()

    def _drain_gathers(b):
        for buf in (ubuf, abuf, bbuf):
            pltpu.make_async_copy(u_hbm.at[pl.ds(0, _C)],
                                  buf.at[pl.ds(b * _C, _C)], sem_g).wait()

    def _compute(b):
        def _row(j, _):
            jj = b * _C + j
            for k in range(_QH):
                msl = pl.ds(k * 16, 16)
                fsl = pl.ds(b * _CH + j * _HH + k * 16, 16)
                osl = pl.ds(j * _HH + k * 16, 16)
                mbuf[j, msl] = jnp.maximum(ubuf[jj, msl] + pmbuf[fsl], 0.0)
                obuf[osl] = ebuf[fsl] + jnp.maximum(
                    abuf[jj, msl] + bbuf[jj, msl] + pebuf[fsl], 0.0)
            return 0

        lax.fori_loop(0, _C, _row, 0)

    def _write(b, i):
        pltpu.sync_copy(mbuf, acc.at[dcp.at[b]], add=True)
        foff = hbase + (ebase + i * _C) * _HH
        pltpu.async_copy(obuf, e_out.at[pl.ds(foff, _CH)], sem_w)

    def _drain_w():
        pltpu.make_async_copy(obuf, e_out.at[pl.ds(0, _CH)], sem_w).wait()

    _fire_idx(0)
    _fire_idx(1)
    _fire_idx(2)
    _fire_streams(0, 0)
    _fire_streams(1, 1)
    _arm_gathers(0, 0)

    def _pair(t, _):
        i0 = 2 * t
        _arm_gathers(1, i0 + 1)

        @pl.when(i0 + 3 < _NCHUNK)
        def _():
            _fire_idx(i0 + 3)

        _drain_streams(0)
        _drain_gathers(0)

        @pl.when(t > 0)
        def _():
            _drain_w()

        _compute(0)
        _write(0, i0)

        @pl.when(i0 + 2 < _NCHUNK)
        def _():
            _fire_streams(0, i0 + 2)
            _arm_gathers(0, i0 + 2)

        @pl.when(i0 + 4 < _NCHUNK)
        def _():
            _fire_idx(i0 + 4)

        _drain_streams(1)
        _drain_gathers(1)
        _drain_w()
        _compute(1)
        _write(1, i0 + 1)

        @pl.when(i0 + 3 < _NCHUNK)
        def _():
            _fire_streams(1, i0 + 3)

        return 0

    lax.fori_loop(0, _NPAIR, _pair, 0)
    _drain_w()
    plsc.subcore_barrier()

    def _flush(t, _):
        r0 = s * _RPT + t * _ZR
        pltpu.sync_copy(acc.at[pl.ds(r0, _ZR)], agg_out.at[c, pl.ds(r0, _ZR)])
        return 0

    lax.fori_loop(0, _RPT // _ZR, _flush, 0)


_sc_layer = functools.partial(
    pl.kernel,
    mesh=_sc_mesh,
    compiler_params=_sc_params,
    out_type=[jax.ShapeDtypeStruct((2 * _E * _HH,), _F32),
              jax.ShapeDtypeStruct((2, _N, _AGGW), _F32)],
    scratch_types=[
        pltpu.VMEM((4, _C), jnp.int32),
        pltpu.VMEM((4, _C), jnp.int32),
        pltpu.VMEM((2, _C), jnp.int32),
        pltpu.VMEM((2, _C), jnp.int32),
        pltpu.VMEM((2, _C), jnp.int32),
        pltpu.VMEM((2 * _C, _HH), _F32),
        pltpu.VMEM((2 * _C, _HH), _F32),
        pltpu.VMEM((2 * _C, _HH), _F32),
        pltpu.VMEM((2 * _CH,), _F32),
        pltpu.VMEM((2 * _CH,), _F32),
        pltpu.VMEM((2 * _CH,), _F32),
        pltpu.VMEM((_CH,), _F32),
        pltpu.VMEM((_C, _AGGW), _F32),
        pltpu.VMEM((_ZR, _AGGW), _F32),
        pltpu.VMEM_SHARED((_N, _AGGW), _F32),
        pltpu.SemaphoreType.DMA,
        pltpu.SemaphoreType.DMA,
        pltpu.SemaphoreType.DMA,
        pltpu.SemaphoreType.DMA,
    ],
)(_sc_layer_body)


_EPW = _E // 32
_C2 = 2000


def _sc_dec_body(src_hbm, dst_hbm, r_hbm, p_hbm, q_hbm, z_out,
                 ptab, qtab, sbuf, dbuf, rbuf, zbuf):
    c = lax.axis_index("c")
    s = lax.axis_index("s")
    w = c * 16 + s
    base = w * _EPW
    pltpu.sync_copy(p_hbm, ptab)
    pltpu.sync_copy(q_hbm, qtab)

    def _chunk(i, _):
        off = base + i * _C2
        pltpu.sync_copy(src_hbm.at[pl.ds(off, _C2)], sbuf)
        pltpu.sync_copy(dst_hbm.at[pl.ds(off, _C2)], dbuf)
        pltpu.sync_copy(r_hbm.at[pl.ds(off, _C2)], rbuf)

        def _vec(t, _):
            sl = pl.ds(t * 16, 16)
            pg = plsc.load_gather(ptab, [sbuf[sl]])
            qg = plsc.load_gather(qtab, [dbuf[sl]])
            zbuf[sl] = jnp.maximum(pg + qg + rbuf[sl], 0.0)
            return 0

        lax.fori_loop(0, _C2 // 16, _vec, 0)
        pltpu.sync_copy(zbuf, z_out.at[pl.ds(off, _C2)])
        return 0

    lax.fori_loop(0, _EPW // _C2, _chunk, 0)


_sc_dec = functools.partial(
    pl.kernel,
    mesh=_sc_mesh,
    compiler_params=_sc_params,
    out_type=jax.ShapeDtypeStruct((_E,), _F32),
    scratch_types=[
        pltpu.VMEM((_N,), _F32),
        pltpu.VMEM((_N,), _F32),
        pltpu.VMEM((_C2,), jnp.int32),
        pltpu.VMEM((_C2,), jnp.int32),
        pltpu.VMEM((_C2,), _F32),
        pltpu.VMEM((_C2,), _F32),
    ],
)(_sc_dec_body)


def kernel(x, edge_index, edge_attr, W_ne, b_ne, W_ee, b_ee, W0x, b0x, W0e,
           b0e, gamma, beta, Wm, bm, We, be, W_dec, b_dec, scale):
    src = edge_index[0]
    dst = edge_index[1]
    r1 = lambda v: v.reshape(1, _H)
    flat = lambda v: v.reshape(2 * _E * _HH)
    tab = lambda v: v.reshape(2 * _N, _HH)
    g = [r1(gamma[0]), r1(gamma[1])]
    bt = [r1(beta[0]), r1(beta[1])]
    wma = [Wm[0, :_H], Wm[1, :_H]]
    wmb = [Wm[0, _H:], Wm[1, _H:]]
    wea = [We[0, :_H], We[1, :_H]]
    web = [We[0, _H:2 * _H], We[1, _H:2 * _H]]
    wec = [We[0, 2 * _H:], We[1, 2 * _H:]]
    bmr = [r1(bm[0]), r1(bm[1])]
    ber = [r1(be[0]), r1(be[1])]
    wdab = W_dec[:2 * _H].reshape(2, _H).T
    wdc = W_dec[2 * _H:]
    bd = b_dec.reshape(1, 1)

    h0, u0, a0, b0 = _tc_node0(x, W_ne, r1(b_ne), W0x, r1(b0x), g[0], bt[0],
                               wma[0], bmr[0], wea[0], ber[0], web[0])
    e0, pm0, pe0 = _tc_edge0(edge_attr, W_ee, r1(b_ee), W0e, r1(b0e), g[0],
                             bt[0], wmb[0], wec[0])
    e1f, aggp0 = _sc_layer(src, dst, flat(e0), flat(pm0), flat(pe0),
                           tab(u0), tab(a0), tab(b0))
    e1 = e1f.reshape(2, _E, _HH)
    h1, u1, a1, b1 = _tc_node_mid(h0, aggp0, g[1], bt[1], wma[1], bmr[1],
                                  wea[1], ber[1], web[1])
    pm1, pe1 = _tc_edge_mid(e1, g[1], bt[1], wmb[1], wec[1])
    e2f, aggp1 = _sc_layer(src, dst, e1f, flat(pm1), flat(pe1),
                           tab(u1), tab(a1), tab(b1))
    e2 = e2f.reshape(2, _E, _HH)
    pq = _tc_node_fin(h1, aggp1, g[0], bt[0], wdab)
    r = _tc_edge_fin(e2, g[0], bt[0], wdc, bd)
    z = _sc_dec(src, dst, r.reshape(_E), pq[:, 0], pq[:, 1])
    return (z * scale).reshape(_E, 1)


# speedup vs baseline: 2.8276x; 1.8443x over previous
import functools

import jax
import jax.numpy as jnp
from jax import lax
from jax.experimental import pallas as pl
from jax.experimental.pallas import tpu as pltpu
from jax.experimental.pallas import tpu_sc as plsc

_N = 10000
_E = 320000
_H = 128
_HH = 64
_F32 = jnp.float32


_ET = 3200
_EG = _E // _ET
_NT = 2000
_NG = _N // _NT
_AGGW = 80

_row_spec_e = pl.BlockSpec((_ET, _H), lambda i: (i, 0))
_row_spec_n = pl.BlockSpec((_NT, _H), lambda i: (i, 0))
_w_spec = pl.BlockSpec((_H, _H), lambda i: (0, 0))
_b_spec = pl.BlockSpec((1, _H), lambda i: (0, 0))
_half_spec_e = pl.BlockSpec((2, _ET, _HH), lambda i: (0, i, 0))
_half_spec_n = pl.BlockSpec((2, _NT, _HH), lambda i: (0, i, 0))
_aggp_spec = pl.BlockSpec((2, _NT, _AGGW), lambda i: (0, i, 0))


def _ln_relu(v, g, b):
    mu = jnp.mean(v, axis=-1, keepdims=True)
    d = v - mu
    var = jnp.mean(d * d, axis=-1, keepdims=True)
    return jnp.maximum(d * lax.rsqrt(var + 1e-5) * g + b, 0.0)


def _dot(a, b):
    return jnp.dot(a, b, preferred_element_type=_F32)


def _split(out_ref, v):
    out_ref[0] = v[:, :_HH]
    out_ref[1] = v[:, _HH:]


def _join(ref):
    v = ref[...]
    return jnp.concatenate([v[0], v[1]], axis=-1)


def _node0_body(x_r, wne_r, bne_r, w0x_r, b0x_r, g_r, bt_r,
                wma_r, bm_r, wea_r, bea_r, web_r,
                h_r, u_r, a_r, b_r):
    h = _dot(x_r[...], wne_r[...]) + bne_r[...]
    h = _dot(h, w0x_r[...]) + b0x_r[...]
    h_r[...] = h
    hn = _ln_relu(h, g_r[...], bt_r[...])
    _split(u_r, _dot(hn, wma_r[...]) + bm_r[...])
    _split(a_r, _dot(hn, wea_r[...]) + bea_r[...])
    _split(b_r, _dot(hn, web_r[...]))


def _tc_node0(x, wne, bne, w0x, b0x, g, bt, wma, bm, wea, bea, web):
    tshp = jax.ShapeDtypeStruct((2, _N, _HH), _F32)
    return pl.pallas_call(
        _node0_body,
        grid=(_NG,),
        in_specs=[_row_spec_n, _w_spec, _b_spec, _w_spec, _b_spec, _b_spec,
                  _b_spec, _w_spec, _b_spec, _w_spec, _b_spec, _w_spec],
        out_specs=[_row_spec_n, _half_spec_n, _half_spec_n, _half_spec_n],
        out_shape=[jax.ShapeDtypeStruct((_N, _H), _F32), tshp, tshp, tshp],
    )(x, wne, bne, w0x, b0x, g, bt, wma, bm, wea, bea, web)


def _apply_agg(h_prev, aggp):
    agg = jnp.concatenate([aggp[0, :, :_HH], aggp[1, :, :_HH]], axis=-1)
    cnt = aggp[0, :, _HH:_HH + 1]
    return h_prev + agg / jnp.maximum(cnt, 1.0)


def _node_mid_body(h_r, aggp_r, g_r, bt_r, wma_r, bm_r, wea_r, bea_r, web_r,
                   hn_out_r, u_r, a_r, b_r):
    h = _apply_agg(h_r[...], aggp_r[...])
    hn_out_r[...] = h
    hn = _ln_relu(h, g_r[...], bt_r[...])
    _split(u_r, _dot(hn, wma_r[...]) + bm_r[...])
    _split(a_r, _dot(hn, wea_r[...]) + bea_r[...])
    _split(b_r, _dot(hn, web_r[...]))


def _tc_node_mid(h, aggp, g, bt, wma, bm, wea, bea, web):
    tshp = jax.ShapeDtypeStruct((2, _N, _HH), _F32)
    return pl.pallas_call(
        _node_mid_body,
        grid=(_NG,),
        in_specs=[_row_spec_n, _aggp_spec, _b_spec, _b_spec, _w_spec, _b_spec,
                  _w_spec, _b_spec, _w_spec],
        out_specs=[_row_spec_n, _half_spec_n, _half_spec_n, _half_spec_n],
        out_shape=[jax.ShapeDtypeStruct((_N, _H), _F32), tshp, tshp, tshp],
    )(h, aggp, g, bt, wma, bm, wea, bea, web)


def _node_fin_body(h_r, aggp_r, g_r, bt_r, wdab_r, pq_r):
    h = _apply_agg(h_r[...], aggp_r[...])
    hf = _ln_relu(h, g_r[...], bt_r[...])
    pq_r[...] = _dot(hf, wdab_r[...])


def _tc_node_fin(h, aggp, g, bt, wdab):
    return pl.pallas_call(
        _node_fin_body,
        grid=(_NG,),
        in_specs=[_row_spec_n, _aggp_spec, _b_spec, _b_spec,
                  pl.BlockSpec((_H, 2), lambda i: (0, 0))],
        out_specs=pl.BlockSpec((_NT, 2), lambda i: (i, 0)),
        out_shape=jax.ShapeDtypeStruct((_N, 2), _F32),
    )(h, aggp, g, bt, wdab)


def _edge0_body(ea_r, wee_r, bee_r, w0e_r, b0e_r, g_r, bt_r, wmb_r, wec_r,
                e_r, pm_r, pe_r):
    e = _dot(ea_r[...], wee_r[...]) + bee_r[...]
    e = _dot(e, w0e_r[...]) + b0e_r[...]
    e_r[...] = e
    gn = _ln_relu(e, g_r[...], bt_r[...])
    pm_r[...] = _dot(gn, wmb_r[...])
    pe_r[...] = _dot(gn, wec_r[...])


def _tc_edge0(ea, wee, bee, w0e, b0e, g, bt, wmb, wec):
    shp = jax.ShapeDtypeStruct((_E, _H), _F32)
    return pl.pallas_call(
        _edge0_body,
        grid=(_EG,),
        in_specs=[pl.BlockSpec((_ET, 16), lambda i: (i, 0)),
                  pl.BlockSpec((16, _H), lambda i: (0, 0)),
                  _b_spec, _w_spec, _b_spec, _b_spec, _b_spec, _w_spec, _w_spec],
        out_specs=[_row_spec_e] * 3,
        out_shape=[shp] * 3,
    )(ea, wee, bee, w0e, b0e, g, bt, wmb, wec)


def _edge_mid_body(e_r, g_r, bt_r, wmb_r, wec_r, pm_r, pe_r):
    gn = _ln_relu(e_r[...], g_r[...], bt_r[...])
    pm_r[...] = _dot(gn, wmb_r[...])
    pe_r[...] = _dot(gn, wec_r[...])


def _tc_edge_mid(e, g, bt, wmb, wec):
    shp = jax.ShapeDtypeStruct((_E, _H), _F32)
    return pl.pallas_call(
        _edge_mid_body,
        grid=(_EG,),
        in_specs=[_row_spec_e, _b_spec, _b_spec, _w_spec, _w_spec],
        out_specs=[_row_spec_e] * 2,
        out_shape=[shp] * 2,
    )(e, g, bt, wmb, wec)


def _edge_fin_body(e_r, g_r, bt_r, wdc_r, bd_r, r_r):
    ef = _ln_relu(e_r[...], g_r[...], bt_r[...])
    r_r[...] = _dot(ef, wdc_r[...]) + bd_r[...]


def _tc_edge_fin(e, g, bt, wdc, bd):
    return pl.pallas_call(
        _edge_fin_body,
        grid=(_EG,),
        in_specs=[_row_spec_e, _b_spec, _b_spec,
                  pl.BlockSpec((_H, 1), lambda i: (0, 0)),
                  pl.BlockSpec((1, 1), lambda i: (0, 0))],
        out_specs=pl.BlockSpec((_ET, 1), lambda i: (i, 0)),
        out_shape=jax.ShapeDtypeStruct((_E, 1), _F32),
    )(e, g, bt, wdc, bd)


_EPS = _E // 16
_C = 80
_NCHUNK = _EPS // _C
_RPT = _N // 16
_ZR = 25
_QH = _HH // 16

_sc_mesh = plsc.VectorSubcoreMesh(core_axis_name="c", subcore_axis_name="s")
_sc_params = pltpu.CompilerParams(use_tc_tiling_on_sc=False,
                                  needs_layout_passes=False)


_CH = _C * _HH
_NPAIR = _NCHUNK // 2


def _sc_layer_body(ei_hbm, e_hbm, pm_hbm, pe_hbm, u_hbm, a_hbm,
                   b_hbm, e_out, agg_out,
                   sidx, didx, gsx, gdx, dcp, ubuf, abuf, bbuf, ebuf, pmbuf,
                   pebuf, obuf, mbuf, zrow, acc, sem_i, sem_l, sem_g, sem_w):
    c = lax.axis_index("c")
    s = lax.axis_index("s")
    ebase = s * _EPS
    chb = c * _HH
    coff = c * _N

    zv = jnp.zeros((16,), _F32)

    def _zfill(j, _):
        for k in range(_AGGW // 16):
            zrow[j, pl.ds(k * 16, 16)] = zv
        return 0

    lax.fori_loop(0, _ZR, _zfill, 0)

    def _zcopy(t, _):
        pltpu.sync_copy(zrow, acc.at[pl.ds(s * _RPT + t * _ZR, _ZR)])
        return 0

    lax.fori_loop(0, _RPT // _ZR, _zcopy, 0)
    plsc.subcore_barrier()

    padv = jnp.where(lax.iota(jnp.int32, 16) == 0,
                     jnp.full((16,), 1.0, _F32), jnp.zeros((16,), _F32))

    def _padfill(j, _):
        mbuf[j, pl.ds(_HH, 16)] = padv
        return 0

    lax.fori_loop(0, _C, _padfill, 0)


    def _fire_idx(i):
        q = lax.rem(i, 4)
        eoff = ebase + i * _C
        pltpu.async_copy(ei_hbm.at[0, pl.ds(eoff, _C)], sidx.at[q], sem_i)
        pltpu.async_copy(ei_hbm.at[1, pl.ds(eoff, _C)], didx.at[q], sem_i)

    def _fire_streams(b, i):
        eoff = ebase + i * _C
        hs = pl.ds(chb, _HH)
        bs = pl.ds(b * _C, _C)
        pltpu.async_copy(e_hbm.at[pl.ds(eoff, _C), hs], ebuf.at[bs], sem_l)
        pltpu.async_copy(pm_hbm.at[pl.ds(eoff, _C), hs], pmbuf.at[bs], sem_l)
        pltpu.async_copy(pe_hbm.at[pl.ds(eoff, _C), hs], pebuf.at[bs], sem_l)

    def _arm_gathers(b, i):
        q = lax.rem(i, 4)
        pltpu.make_async_copy(ei_hbm.at[0, pl.ds(0, _C)], sidx.at[0],
                              sem_i).wait()
        pltpu.make_async_copy(ei_hbm.at[0, pl.ds(0, _C)], didx.at[0],
                              sem_i).wait()

        def _shift(j, _):
            sl = pl.ds(j * 16, 16)
            gsx[b, sl] = sidx[q, sl] + coff
            gdx[b, sl] = didx[q, sl] + coff
            dcp[b, sl] = didx[q, sl]
            return 0

        lax.fori_loop(0, _C // 16, _shift, 0)
        pltpu.async_copy(u_hbm.at[gsx.at[b]], ubuf.at[pl.ds(b * _C, _C)],
                         sem_g)
        pltpu.async_copy(a_hbm.at[gsx.at[b]], abuf.at[pl.ds(b * _C, _C)],
                         sem_g)
        pltpu.async_copy(b_hbm.at[gdx.at[b]], bbuf.at[pl.ds(b * _C, _C)],
                         sem_g)

    def _drain_streams(b):
        for buf in (ebuf, pmbuf, pebuf):
            pltpu.make_async_copy(e_hbm.at[pl.ds(0, _C), pl.ds(0, _HH)],
                                  buf.at[pl.ds(b * _C, _C)], sem_l).wait()

    def _drain_gathers(b):
        for buf in (ubuf, abuf, bbuf):
            pltpu.make_async_copy(u_hbm.at[pl.ds(0, _C)],
                                  buf.at[pl.ds(b * _C, _C)], sem_g).wait()

    def _compute(b):
        def _row(j, _):
            jj = b * _C + j
            for k in range(_QH):
                msl = pl.ds(k * 16, 16)
                mbuf[j, msl] = jnp.maximum(ubuf[jj, msl] + pmbuf[jj, msl], 0.0)
                obuf[j, msl] = ebuf[jj, msl] + jnp.maximum(
                    abuf[jj, msl] + bbuf[jj, msl] + pebuf[jj, msl], 0.0)
            return 0

        lax.fori_loop(0, _C, _row, 0)

    def _write(b, i):
        pltpu.sync_copy(mbuf, acc.at[dcp.at[b]], add=True)
        eoff = ebase + i * _C
        pltpu.async_copy(obuf, e_out.at[pl.ds(eoff, _C), pl.ds(chb, _HH)],
                         sem_w)

    def _drain_w():
        pltpu.make_async_copy(obuf, e_out.at[pl.ds(0, _C), pl.ds(0, _HH)],
                              sem_w).wait()

    _fire_idx(0)
    _fire_idx(1)
    _fire_idx(2)
    _fire_streams(0, 0)
    _fire_streams(1, 1)
    _arm_gathers(0, 0)

    def _pair(t, _):
        i0 = 2 * t
        _arm_gathers(1, i0 + 1)

        @pl.when(i0 + 3 < _NCHUNK)
        def _():
            _fire_idx(i0 + 3)

        _drain_streams(0)
        _drain_gathers(0)

        @pl.when(t > 0)
        def _():
            _drain_w()

        _compute(0)
        _write(0, i0)

        @pl.when(i0 + 2 < _NCHUNK)
        def _():
            _fire_streams(0, i0 + 2)
            _arm_gathers(0, i0 + 2)

        @pl.when(i0 + 4 < _NCHUNK)
        def _():
            _fire_idx(i0 + 4)

        _drain_streams(1)
        _drain_gathers(1)
        _drain_w()
        _compute(1)
        _write(1, i0 + 1)

        @pl.when(i0 + 3 < _NCHUNK)
        def _():
            _fire_streams(1, i0 + 3)

        return 0

    lax.fori_loop(0, _NPAIR, _pair, 0)
    _drain_w()
    plsc.subcore_barrier()

    def _flush(t, _):
        r0 = s * _RPT + t * _ZR
        pltpu.sync_copy(acc.at[pl.ds(r0, _ZR)], agg_out.at[c, pl.ds(r0, _ZR)])
        return 0

    lax.fori_loop(0, _RPT // _ZR, _flush, 0)


_sc_layer = functools.partial(
    pl.kernel,
    mesh=_sc_mesh,
    compiler_params=_sc_params,
    out_type=[jax.ShapeDtypeStruct((_E, _H), _F32),
              jax.ShapeDtypeStruct((2, _N, _AGGW), _F32)],
    scratch_types=[
        pltpu.VMEM((4, _C), jnp.int32),
        pltpu.VMEM((4, _C), jnp.int32),
        pltpu.VMEM((2, _C), jnp.int32),
        pltpu.VMEM((2, _C), jnp.int32),
        pltpu.VMEM((2, _C), jnp.int32),
        pltpu.VMEM((2 * _C, _HH), _F32),
        pltpu.VMEM((2 * _C, _HH), _F32),
        pltpu.VMEM((2 * _C, _HH), _F32),
        pltpu.VMEM((2 * _C, _HH), _F32),
        pltpu.VMEM((2 * _C, _HH), _F32),
        pltpu.VMEM((2 * _C, _HH), _F32),
        pltpu.VMEM((_C, _HH), _F32),
        pltpu.VMEM((_C, _AGGW), _F32),
        pltpu.VMEM((_ZR, _AGGW), _F32),
        pltpu.VMEM_SHARED((_N, _AGGW), _F32),
        pltpu.SemaphoreType.DMA,
        pltpu.SemaphoreType.DMA,
        pltpu.SemaphoreType.DMA,
        pltpu.SemaphoreType.DMA,
    ],
)(_sc_layer_body)


_EPW = _E // 32
_C2 = 2000


def _sc_dec_body(ei_hbm, r_hbm, p_hbm, q_hbm, z_out,
                 ptab, qtab, sbuf, dbuf, rbuf, zbuf):
    c = lax.axis_index("c")
    s = lax.axis_index("s")
    w = c * 16 + s
    base = w * _EPW
    pltpu.sync_copy(p_hbm, ptab)
    pltpu.sync_copy(q_hbm, qtab)

    def _chunk(i, _):
        off = base + i * _C2
        pltpu.sync_copy(ei_hbm.at[0, pl.ds(off, _C2)], sbuf)
        pltpu.sync_copy(ei_hbm.at[1, pl.ds(off, _C2)], dbuf)
        pltpu.sync_copy(r_hbm.at[pl.ds(off, _C2)], rbuf)

        def _vec(t, _):
            sl = pl.ds(t * 16, 16)
            pg = plsc.load_gather(ptab, [sbuf[sl]])
            qg = plsc.load_gather(qtab, [dbuf[sl]])
            zbuf[sl] = jnp.maximum(pg + qg + rbuf[sl], 0.0)
            return 0

        lax.fori_loop(0, _C2 // 16, _vec, 0)
        pltpu.sync_copy(zbuf, z_out.at[pl.ds(off, _C2)])
        return 0

    lax.fori_loop(0, _EPW // _C2, _chunk, 0)


_sc_dec = functools.partial(
    pl.kernel,
    mesh=_sc_mesh,
    compiler_params=_sc_params,
    out_type=jax.ShapeDtypeStruct((_E,), _F32),
    scratch_types=[
        pltpu.VMEM((_N,), _F32),
        pltpu.VMEM((_N,), _F32),
        pltpu.VMEM((_C2,), jnp.int32),
        pltpu.VMEM((_C2,), jnp.int32),
        pltpu.VMEM((_C2,), _F32),
        pltpu.VMEM((_C2,), _F32),
    ],
)(_sc_dec_body)


def kernel(x, edge_index, edge_attr, W_ne, b_ne, W_ee, b_ee, W0x, b0x, W0e,
           b0e, gamma, beta, Wm, bm, We, be, W_dec, b_dec, scale):
    r1 = lambda v: v.reshape(1, _H)
    tab = lambda v: v.reshape(2 * _N, _HH)
    g = [r1(gamma[0]), r1(gamma[1])]
    bt = [r1(beta[0]), r1(beta[1])]
    wma = [Wm[0, :_H], Wm[1, :_H]]
    wmb = [Wm[0, _H:], Wm[1, _H:]]
    wea = [We[0, :_H], We[1, :_H]]
    web = [We[0, _H:2 * _H], We[1, _H:2 * _H]]
    wec = [We[0, 2 * _H:], We[1, 2 * _H:]]
    bmr = [r1(bm[0]), r1(bm[1])]
    ber = [r1(be[0]), r1(be[1])]
    wdab = W_dec[:2 * _H].reshape(2, _H).T
    wdc = W_dec[2 * _H:]
    bd = b_dec.reshape(1, 1)

    h0, u0, a0, b0 = _tc_node0(x, W_ne, r1(b_ne), W0x, r1(b0x), g[0], bt[0],
                               wma[0], bmr[0], wea[0], ber[0], web[0])
    e0, pm0, pe0 = _tc_edge0(edge_attr, W_ee, r1(b_ee), W0e, r1(b0e), g[0],
                             bt[0], wmb[0], wec[0])
    e1, aggp0 = _sc_layer(edge_index, e0, pm0, pe0, tab(u0), tab(a0), tab(b0))
    h1, u1, a1, b1 = _tc_node_mid(h0, aggp0, g[1], bt[1], wma[1], bmr[1],
                                  wea[1], ber[1], web[1])
    pm1, pe1 = _tc_edge_mid(e1, g[1], bt[1], wmb[1], wec[1])
    e2, aggp1 = _sc_layer(edge_index, e1, pm1, pe1, tab(u1), tab(a1), tab(b1))
    pq = _tc_node_fin(h1, aggp1, g[0], bt[0], wdab)
    r = _tc_edge_fin(e2, g[0], bt[0], wdc, bd)
    z = _sc_dec(edge_index, r.reshape(_E), pq[:, 0], pq[:, 1])
    return (z * scale).reshape(_E, 1)


# speedup vs baseline: 3.6280x; 1.2831x over previous
import functools

import jax
import jax.numpy as jnp
from jax import lax
from jax.experimental import pallas as pl
from jax.experimental.pallas import tpu as pltpu
from jax.experimental.pallas import tpu_sc as plsc

_N = 10000
_E = 320000
_H = 128
_HH = 64
_F32 = jnp.float32


_ET = 3200
_EG = _E // _ET
_NT = 2000
_NG = _N // _NT
_AGGW = 80

_row_spec_e = pl.BlockSpec((_ET, _H), lambda i: (i, 0))
_row_spec_n = pl.BlockSpec((_NT, _H), lambda i: (i, 0))
_w_spec = pl.BlockSpec((_H, _H), lambda i: (0, 0))
_b_spec = pl.BlockSpec((1, _H), lambda i: (0, 0))
_half_spec_e = pl.BlockSpec((2, _ET, _HH), lambda i: (0, i, 0))
_half_spec_n = pl.BlockSpec((2, _NT, _HH), lambda i: (0, i, 0))
_aggp_spec = pl.BlockSpec((2, _NT, _AGGW), lambda i: (0, i, 0))


def _ln_relu(v, g, b):
    mu = jnp.mean(v, axis=-1, keepdims=True)
    d = v - mu
    var = jnp.mean(d * d, axis=-1, keepdims=True)
    return jnp.maximum(d * lax.rsqrt(var + 1e-5) * g + b, 0.0)


def _dot(a, b):
    return jnp.dot(a, b, preferred_element_type=_F32)


def _split(out_ref, v):
    out_ref[0] = v[:, :_HH]
    out_ref[1] = v[:, _HH:]


def _join(ref):
    v = ref[...]
    return jnp.concatenate([v[0], v[1]], axis=-1)


def _node0_body(x_r, wne_r, bne_r, w0x_r, b0x_r, g_r, bt_r,
                wma_r, bm_r, wea_r, bea_r, web_r,
                h_r, u_r, a_r, b_r):
    h = _dot(x_r[...], wne_r[...]) + bne_r[...]
    h = _dot(h, w0x_r[...]) + b0x_r[...]
    h_r[...] = h
    hn = _ln_relu(h, g_r[...], bt_r[...])
    _split(u_r, _dot(hn, wma_r[...]) + bm_r[...])
    _split(a_r, _dot(hn, wea_r[...]) + bea_r[...])
    _split(b_r, _dot(hn, web_r[...]))


def _tc_node0(x, wne, bne, w0x, b0x, g, bt, wma, bm, wea, bea, web):
    tshp = jax.ShapeDtypeStruct((2, _N, _HH), _F32)
    return pl.pallas_call(
        _node0_body,
        grid=(_NG,),
        in_specs=[_row_spec_n, _w_spec, _b_spec, _w_spec, _b_spec, _b_spec,
                  _b_spec, _w_spec, _b_spec, _w_spec, _b_spec, _w_spec],
        out_specs=[_row_spec_n, _half_spec_n, _half_spec_n, _half_spec_n],
        out_shape=[jax.ShapeDtypeStruct((_N, _H), _F32), tshp, tshp, tshp],
    )(x, wne, bne, w0x, b0x, g, bt, wma, bm, wea, bea, web)


def _apply_agg(h_prev, aggp):
    agg = jnp.concatenate([aggp[0, :, :_HH], aggp[1, :, :_HH]], axis=-1)
    cnt = aggp[0, :, _HH:_HH + 1]
    return h_prev + agg / jnp.maximum(cnt, 1.0)


def _node_mid_body(h_r, aggp_r, g_r, bt_r, wma_r, bm_r, wea_r, bea_r, web_r,
                   hn_out_r, u_r, a_r, b_r):
    h = _apply_agg(h_r[...], aggp_r[...])
    hn_out_r[...] = h
    hn = _ln_relu(h, g_r[...], bt_r[...])
    _split(u_r, _dot(hn, wma_r[...]) + bm_r[...])
    _split(a_r, _dot(hn, wea_r[...]) + bea_r[...])
    _split(b_r, _dot(hn, web_r[...]))


def _tc_node_mid(h, aggp, g, bt, wma, bm, wea, bea, web):
    tshp = jax.ShapeDtypeStruct((2, _N, _HH), _F32)
    return pl.pallas_call(
        _node_mid_body,
        grid=(_NG,),
        in_specs=[_row_spec_n, _aggp_spec, _b_spec, _b_spec, _w_spec, _b_spec,
                  _w_spec, _b_spec, _w_spec],
        out_specs=[_row_spec_n, _half_spec_n, _half_spec_n, _half_spec_n],
        out_shape=[jax.ShapeDtypeStruct((_N, _H), _F32), tshp, tshp, tshp],
    )(h, aggp, g, bt, wma, bm, wea, bea, web)


def _node_fin_body(h_r, aggp_r, g_r, bt_r, wdab_r, pq_r):
    h = _apply_agg(h_r[...], aggp_r[...])
    hf = _ln_relu(h, g_r[...], bt_r[...])
    pq_r[...] = _dot(hf, wdab_r[...])


def _tc_node_fin(h, aggp, g, bt, wdab):
    return pl.pallas_call(
        _node_fin_body,
        grid=(_NG,),
        in_specs=[_row_spec_n, _aggp_spec, _b_spec, _b_spec,
                  pl.BlockSpec((_H, 2), lambda i: (0, 0))],
        out_specs=pl.BlockSpec((_NT, 2), lambda i: (i, 0)),
        out_shape=jax.ShapeDtypeStruct((_N, 2), _F32),
    )(h, aggp, g, bt, wdab)


def _edge0_body(ea_r, wee_r, bee_r, w0e_r, b0e_r, g_r, bt_r, wmb_r, wec_r,
                e_r, pm_r, pe_r):
    e = _dot(ea_r[...], wee_r[...]) + bee_r[...]
    e = _dot(e, w0e_r[...]) + b0e_r[...]
    e_r[...] = e
    gn = _ln_relu(e, g_r[...], bt_r[...])
    pm_r[...] = _dot(gn, wmb_r[...])
    pe_r[...] = _dot(gn, wec_r[...])


def _tc_edge0(ea, wee, bee, w0e, b0e, g, bt, wmb, wec):
    shp = jax.ShapeDtypeStruct((_E, _H), _F32)
    return pl.pallas_call(
        _edge0_body,
        grid=(_EG,),
        in_specs=[pl.BlockSpec((_ET, 16), lambda i: (i, 0)),
                  pl.BlockSpec((16, _H), lambda i: (0, 0)),
                  _b_spec, _w_spec, _b_spec, _b_spec, _b_spec, _w_spec, _w_spec],
        out_specs=[_row_spec_e] * 3,
        out_shape=[shp] * 3,
    )(ea, wee, bee, w0e, b0e, g, bt, wmb, wec)


def _edge_mid_body(e_r, g_r, bt_r, wmb_r, wec_r, pm_r, pe_r):
    gn = _ln_relu(e_r[...], g_r[...], bt_r[...])
    pm_r[...] = _dot(gn, wmb_r[...])
    pe_r[...] = _dot(gn, wec_r[...])


def _tc_edge_mid(e, g, bt, wmb, wec):
    shp = jax.ShapeDtypeStruct((_E, _H), _F32)
    return pl.pallas_call(
        _edge_mid_body,
        grid=(_EG,),
        in_specs=[_row_spec_e, _b_spec, _b_spec, _w_spec, _w_spec],
        out_specs=[_row_spec_e] * 2,
        out_shape=[shp] * 2,
    )(e, g, bt, wmb, wec)


def _edge_fin_body(e_r, g_r, bt_r, wdc_r, bd_r, r_r):
    ef = _ln_relu(e_r[...], g_r[...], bt_r[...])
    r_r[...] = _dot(ef, wdc_r[...]) + bd_r[...]


def _tc_edge_fin(e, g, bt, wdc, bd):
    return pl.pallas_call(
        _edge_fin_body,
        grid=(_EG,),
        in_specs=[_row_spec_e, _b_spec, _b_spec,
                  pl.BlockSpec((_H, 1), lambda i: (0, 0)),
                  pl.BlockSpec((1, 1), lambda i: (0, 0))],
        out_specs=pl.BlockSpec((_ET, 1), lambda i: (i, 0)),
        out_shape=jax.ShapeDtypeStruct((_E, 1), _F32),
    )(e, g, bt, wdc, bd)


_EPS = _E // 16
_C = 80
_NCHUNK = _EPS // _C
_RPT = _N // 16
_ZR = 25
_QH = _HH // 16

_sc_mesh = plsc.VectorSubcoreMesh(core_axis_name="c", subcore_axis_name="s")
_sc_params = pltpu.CompilerParams(use_tc_tiling_on_sc=False,
                                  needs_layout_passes=False)


_CH = _C * _HH
_NPAIR = _NCHUNK // 2


def _sc_e_body(ei_hbm, e_hbm, pe_hbm, a_hbm, b_hbm, e_out,
               sidx, didx, gsx, gdx, abuf, bbuf, ebuf, pebuf, obuf,
               sem_i, sem_l, sem_g, sem_w):
    c = lax.axis_index("c")
    s = lax.axis_index("s")
    ebase = s * _EPS
    chb = c * _HH
    coff = c * _N

    def _fire_idx(i):
        q = lax.rem(i, 4)
        eoff = ebase + i * _C
        pltpu.async_copy(ei_hbm.at[0, pl.ds(eoff, _C)], sidx.at[q], sem_i)
        pltpu.async_copy(ei_hbm.at[1, pl.ds(eoff, _C)], didx.at[q], sem_i)

    def _fire_streams(b, i):
        eoff = ebase + i * _C
        hs = pl.ds(chb, _HH)
        bs = pl.ds(b * _C, _C)
        pltpu.async_copy(e_hbm.at[pl.ds(eoff, _C), hs], ebuf.at[bs], sem_l)
        pltpu.async_copy(pe_hbm.at[pl.ds(eoff, _C), hs], pebuf.at[bs], sem_l)

    def _arm_gathers(b, i):
        q = lax.rem(i, 4)
        pltpu.make_async_copy(ei_hbm.at[0, pl.ds(0, _C)], sidx.at[0],
                              sem_i).wait()
        pltpu.make_async_copy(ei_hbm.at[0, pl.ds(0, _C)], didx.at[0],
                              sem_i).wait()

        def _shift(j, _):
            sl = pl.ds(j * 16, 16)
            gsx[b, sl] = sidx[q, sl] + coff
            gdx[b, sl] = didx[q, sl] + coff
            return 0

        lax.fori_loop(0, _C // 16, _shift, 0)
        pltpu.async_copy(a_hbm.at[gsx.at[b]], abuf.at[pl.ds(b * _C, _C)],
                         sem_g)
        pltpu.async_copy(b_hbm.at[gdx.at[b]], bbuf.at[pl.ds(b * _C, _C)],
                         sem_g)

    def _drain_streams(b):
        for buf in (ebuf, pebuf):
            pltpu.make_async_copy(e_hbm.at[pl.ds(0, _C), pl.ds(0, _HH)],
                                  buf.at[pl.ds(b * _C, _C)], sem_l).wait()

    def _drain_gathers(b):
        for buf in (abuf, bbuf):
            pltpu.make_async_copy(a_hbm.at[pl.ds(0, _C)],
                                  buf.at[pl.ds(b * _C, _C)], sem_g).wait()

    def _compute(b):
        def _row(j, _):
            jj = b * _C + j
            for k in range(_QH):
                msl = pl.ds(k * 16, 16)
                obuf[j, msl] = ebuf[jj, msl] + jnp.maximum(
                    abuf[jj, msl] + bbuf[jj, msl] + pebuf[jj, msl], 0.0)
            return 0

        lax.fori_loop(0, _C, _row, 0)

    def _write(b, i):
        eoff = ebase + i * _C
        pltpu.async_copy(obuf, e_out.at[pl.ds(eoff, _C), pl.ds(chb, _HH)],
                         sem_w)

    def _drain_w():
        pltpu.make_async_copy(obuf, e_out.at[pl.ds(0, _C), pl.ds(0, _HH)],
                              sem_w).wait()

    _fire_idx(0)
    _fire_idx(1)
    _fire_idx(2)
    _fire_streams(0, 0)
    _fire_streams(1, 1)
    _arm_gathers(0, 0)

    def _pair(t, _):
        i0 = 2 * t
        _arm_gathers(1, i0 + 1)

        @pl.when(i0 + 3 < _NCHUNK)
        def _():
            _fire_idx(i0 + 3)

        _drain_streams(0)
        _drain_gathers(0)

        @pl.when(t > 0)
        def _():
            _drain_w()

        _compute(0)
        _write(0, i0)

        @pl.when(i0 + 2 < _NCHUNK)
        def _():
            _fire_streams(0, i0 + 2)
            _arm_gathers(0, i0 + 2)

        @pl.when(i0 + 4 < _NCHUNK)
        def _():
            _fire_idx(i0 + 4)

        _drain_streams(1)
        _drain_gathers(1)
        _drain_w()
        _compute(1)
        _write(1, i0 + 1)

        @pl.when(i0 + 3 < _NCHUNK)
        def _():
            _fire_streams(1, i0 + 3)

        return 0

    lax.fori_loop(0, _NPAIR, _pair, 0)
    _drain_w()


_sc_e = functools.partial(
    pl.kernel,
    mesh=_sc_mesh,
    compiler_params=_sc_params,
    out_type=jax.ShapeDtypeStruct((_E, _H), _F32),
    scratch_types=[
        pltpu.VMEM((4, _C), jnp.int32),
        pltpu.VMEM((4, _C), jnp.int32),
        pltpu.VMEM((2, _C), jnp.int32),
        pltpu.VMEM((2, _C), jnp.int32),
        pltpu.VMEM((2 * _C, _HH), _F32),
        pltpu.VMEM((2 * _C, _HH), _F32),
        pltpu.VMEM((2 * _C, _HH), _F32),
        pltpu.VMEM((2 * _C, _HH), _F32),
        pltpu.VMEM((_C, _HH), _F32),
        pltpu.SemaphoreType.DMA,
        pltpu.SemaphoreType.DMA,
        pltpu.SemaphoreType.DMA,
        pltpu.SemaphoreType.DMA,
    ],
)(_sc_e_body)


def _sc_m_body(ei_hbm, pm_hbm, u_hbm, edep_hbm, agg_out,
               sidx, didx, gsx, dcp, ubuf, pmbuf, mbuf, zrow, acc,
               sem_i, sem_l, sem_g, sem_s):
    del edep_hbm
    c = lax.axis_index("c")
    s = lax.axis_index("s")
    ebase = s * _EPS
    chb = c * _HH
    coff = c * _N

    zv = jnp.zeros((16,), _F32)

    def _zfill(j, _):
        for k in range(_AGGW // 16):
            zrow[j, pl.ds(k * 16, 16)] = zv
        return 0

    lax.fori_loop(0, _ZR, _zfill, 0)

    def _zcopy(t, _):
        pltpu.sync_copy(zrow, acc.at[pl.ds(s * _RPT + t * _ZR, _ZR)])
        return 0

    lax.fori_loop(0, _RPT // _ZR, _zcopy, 0)
    plsc.subcore_barrier()

    padv = jnp.where(lax.iota(jnp.int32, 16) == 0,
                     jnp.full((16,), 1.0, _F32), jnp.zeros((16,), _F32))

    def _padfill(j, _):
        mbuf[j, pl.ds(_HH, 16)] = padv
        return 0

    lax.fori_loop(0, 2 * _C, _padfill, 0)

    def _fire_idx(i):
        q = lax.rem(i, 4)
        eoff = ebase + i * _C
        pltpu.async_copy(ei_hbm.at[0, pl.ds(eoff, _C)], sidx.at[q], sem_i)
        pltpu.async_copy(ei_hbm.at[1, pl.ds(eoff, _C)], didx.at[q], sem_i)

    def _fire_streams(b, i):
        eoff = ebase + i * _C
        pltpu.async_copy(pm_hbm.at[pl.ds(eoff, _C), pl.ds(chb, _HH)],
                         pmbuf.at[pl.ds(b * _C, _C)], sem_l)

    def _arm_gathers(b, i):
        q = lax.rem(i, 4)
        pltpu.make_async_copy(ei_hbm.at[0, pl.ds(0, _C)], sidx.at[0],
                              sem_i).wait()
        pltpu.make_async_copy(ei_hbm.at[0, pl.ds(0, _C)], didx.at[0],
                              sem_i).wait()

        def _shift(j, _):
            sl = pl.ds(j * 16, 16)
            gsx[b, sl] = sidx[q, sl] + coff
            dcp[q, sl] = didx[q, sl]
            return 0

        lax.fori_loop(0, _C // 16, _shift, 0)
        pltpu.async_copy(u_hbm.at[gsx.at[b]], ubuf.at[pl.ds(b * _C, _C)],
                         sem_g)

    def _drain_streams(b):
        pltpu.make_async_copy(pm_hbm.at[pl.ds(0, _C), pl.ds(0, _HH)],
                              pmbuf.at[pl.ds(b * _C, _C)], sem_l).wait()

    def _drain_gathers(b):
        pltpu.make_async_copy(u_hbm.at[pl.ds(0, _C)],
                              ubuf.at[pl.ds(b * _C, _C)], sem_g).wait()

    def _compute(b):
        def _row(j, _):
            jj = b * _C + j
            for k in range(_QH):
                msl = pl.ds(k * 16, 16)
                mbuf[jj, msl] = jnp.maximum(ubuf[jj, msl] + pmbuf[jj, msl],
                                            0.0)
            return 0

        lax.fori_loop(0, _C, _row, 0)

    def _write(b, i):
        q = lax.rem(i, 4)
        pltpu.async_copy(mbuf.at[pl.ds(b * _C, _C)], acc.at[dcp.at[q]],
                         sem_s, add=True)

    def _drain_s(b):
        pltpu.make_async_copy(mbuf.at[pl.ds(b * _C, _C)],
                              acc.at[pl.ds(0, _C)], sem_s).wait()

    _fire_idx(0)
    _fire_idx(1)
    _fire_idx(2)
    _fire_streams(0, 0)
    _fire_streams(1, 1)
    _arm_gathers(0, 0)

    def _pair(t, _):
        i0 = 2 * t
        _arm_gathers(1, i0 + 1)

        @pl.when(i0 + 3 < _NCHUNK)
        def _():
            _fire_idx(i0 + 3)

        _drain_streams(0)
        _drain_gathers(0)

        @pl.when(t > 0)
        def _():
            _drain_s(0)

        _compute(0)
        _write(0, i0)

        @pl.when(i0 + 2 < _NCHUNK)
        def _():
            _fire_streams(0, i0 + 2)
            _arm_gathers(0, i0 + 2)

        @pl.when(i0 + 4 < _NCHUNK)
        def _():
            _fire_idx(i0 + 4)

        _drain_streams(1)
        _drain_gathers(1)

        @pl.when(t > 0)
        def _():
            _drain_s(1)

        _compute(1)
        _write(1, i0 + 1)

        @pl.when(i0 + 3 < _NCHUNK)
        def _():
            _fire_streams(1, i0 + 3)

        return 0

    lax.fori_loop(0, _NPAIR, _pair, 0)
    _drain_s(0)
    _drain_s(1)
    plsc.subcore_barrier()

    def _flush(t, _):
        r0 = s * _RPT + t * _ZR
        pltpu.sync_copy(acc.at[pl.ds(r0, _ZR)], agg_out.at[c, pl.ds(r0, _ZR)])
        return 0

    lax.fori_loop(0, _RPT // _ZR, _flush, 0)


_sc_m = functools.partial(
    pl.kernel,
    mesh=_sc_mesh,
    compiler_params=_sc_params,
    out_type=jax.ShapeDtypeStruct((2, _N, _AGGW), _F32),
    scratch_types=[
        pltpu.VMEM((4, _C), jnp.int32),
        pltpu.VMEM((4, _C), jnp.int32),
        pltpu.VMEM((2, _C), jnp.int32),
        pltpu.VMEM((4, _C), jnp.int32),
        pltpu.VMEM((2 * _C, _HH), _F32),
        pltpu.VMEM((2 * _C, _HH), _F32),
        pltpu.VMEM((2 * _C, _AGGW), _F32),
        pltpu.VMEM((_ZR, _AGGW), _F32),
        pltpu.VMEM_SHARED((_N, _AGGW), _F32),
        pltpu.SemaphoreType.DMA,
        pltpu.SemaphoreType.DMA,
        pltpu.SemaphoreType.DMA,
        pltpu.SemaphoreType.DMA,
    ],
)(_sc_m_body)


_EPW = _E // 32
_C2 = 2000


def _sc_dec_body(ei_hbm, r_hbm, p_hbm, q_hbm, z_out,
                 ptab, qtab, sbuf, dbuf, rbuf, zbuf):
    c = lax.axis_index("c")
    s = lax.axis_index("s")
    w = c * 16 + s
    base = w * _EPW
    pltpu.sync_copy(p_hbm, ptab)
    pltpu.sync_copy(q_hbm, qtab)

    def _chunk(i, _):
        off = base + i * _C2
        pltpu.sync_copy(ei_hbm.at[0, pl.ds(off, _C2)], sbuf)
        pltpu.sync_copy(ei_hbm.at[1, pl.ds(off, _C2)], dbuf)
        pltpu.sync_copy(r_hbm.at[pl.ds(off, _C2)], rbuf)

        def _vec(t, _):
            sl = pl.ds(t * 16, 16)
            pg = plsc.load_gather(ptab, [sbuf[sl]])
            qg = plsc.load_gather(qtab, [dbuf[sl]])
            zbuf[sl] = jnp.maximum(pg + qg + rbuf[sl], 0.0)
            return 0

        lax.fori_loop(0, _C2 // 16, _vec, 0)
        pltpu.sync_copy(zbuf, z_out.at[pl.ds(off, _C2)])
        return 0

    lax.fori_loop(0, _EPW // _C2, _chunk, 0)


_sc_dec = functools.partial(
    pl.kernel,
    mesh=_sc_mesh,
    compiler_params=_sc_params,
    out_type=jax.ShapeDtypeStruct((_E,), _F32),
    scratch_types=[
        pltpu.VMEM((_N,), _F32),
        pltpu.VMEM((_N,), _F32),
        pltpu.VMEM((_C2,), jnp.int32),
        pltpu.VMEM((_C2,), jnp.int32),
        pltpu.VMEM((_C2,), _F32),
        pltpu.VMEM((_C2,), _F32),
    ],
)(_sc_dec_body)


def kernel(x, edge_index, edge_attr, W_ne, b_ne, W_ee, b_ee, W0x, b0x, W0e,
           b0e, gamma, beta, Wm, bm, We, be, W_dec, b_dec, scale):
    r1 = lambda v: v.reshape(1, _H)
    tab = lambda v: v.reshape(2 * _N, _HH)
    g = [r1(gamma[0]), r1(gamma[1])]
    bt = [r1(beta[0]), r1(beta[1])]
    wma = [Wm[0, :_H], Wm[1, :_H]]
    wmb = [Wm[0, _H:], Wm[1, _H:]]
    wea = [We[0, :_H], We[1, :_H]]
    web = [We[0, _H:2 * _H], We[1, _H:2 * _H]]
    wec = [We[0, 2 * _H:], We[1, 2 * _H:]]
    bmr = [r1(bm[0]), r1(bm[1])]
    ber = [r1(be[0]), r1(be[1])]
    wdab = W_dec[:2 * _H].reshape(2, _H).T
    wdc = W_dec[2 * _H:]
    bd = b_dec.reshape(1, 1)

    h0, u0, a0, b0 = _tc_node0(x, W_ne, r1(b_ne), W0x, r1(b0x), g[0], bt[0],
                               wma[0], bmr[0], wea[0], ber[0], web[0])
    e0, pm0, pe0 = _tc_edge0(edge_attr, W_ee, r1(b_ee), W0e, r1(b0e), g[0],
                             bt[0], wmb[0], wec[0])
    e1 = _sc_e(edge_index, e0, pe0, tab(a0), tab(b0))
    aggp0 = _sc_m(edge_index, pm0, tab(u0), e1)
    pm1, pe1 = _tc_edge_mid(e1, g[1], bt[1], wmb[1], wec[1])
    h1, u1, a1, b1 = _tc_node_mid(h0, aggp0, g[1], bt[1], wma[1], bmr[1],
                                  wea[1], ber[1], web[1])
    e2 = _sc_e(edge_index, e1, pe1, tab(a1), tab(b1))
    aggp1 = _sc_m(edge_index, pm1, tab(u1), e2)
    r = _tc_edge_fin(e2, g[0], bt[0], wdc, bd)
    pq = _tc_node_fin(h1, aggp1, g[0], bt[0], wdab)
    z = _sc_dec(edge_index, r.reshape(_E), pq[:, 0], pq[:, 1])
    return (z * scale).reshape(_E, 1)


# speedup vs baseline: 4.0599x; 1.1191x over previous
import functools

import jax
import jax.numpy as jnp
from jax import lax
from jax.experimental import pallas as pl
from jax.experimental.pallas import tpu as pltpu
from jax.experimental.pallas import tpu_sc as plsc

_N = 10000
_E = 320000
_H = 128
_HH = 64
_F32 = jnp.float32


_ET = 3200
_EG = _E // _ET
_NT = 2000
_NG = _N // _NT
_AGGW = 80

_row_spec_e = pl.BlockSpec((_ET, _H), lambda i: (i, 0))
_row_spec_n = pl.BlockSpec((_NT, _H), lambda i: (i, 0))
_w_spec = pl.BlockSpec((_H, _H), lambda i: (0, 0))
_b_spec = pl.BlockSpec((1, _H), lambda i: (0, 0))
_half_spec_e = pl.BlockSpec((2, _ET, _HH), lambda i: (0, i, 0))
_half_spec_n = pl.BlockSpec((2, _NT, _HH), lambda i: (0, i, 0))
_aggp_spec = pl.BlockSpec((2, _NT, _AGGW), lambda i: (0, i, 0))


def _ln_relu(v, g, b):
    mu = jnp.mean(v, axis=-1, keepdims=True)
    d = v - mu
    var = jnp.mean(d * d, axis=-1, keepdims=True)
    return jnp.maximum(d * lax.rsqrt(var + 1e-5) * g + b, 0.0)


def _dot(a, b):
    return jnp.dot(a, b, preferred_element_type=_F32)


def _split(out_ref, v):
    out_ref[0] = v[:, :_HH]
    out_ref[1] = v[:, _HH:]


def _join(ref):
    v = ref[...]
    return jnp.concatenate([v[0], v[1]], axis=-1)


def _node0_body(x_r, wne_r, bne_r, w0x_r, b0x_r, g_r, bt_r,
                wma_r, bm_r, wea_r, bea_r, web_r,
                h_r, u_r, a_r, b_r):
    h = _dot(x_r[...], wne_r[...]) + bne_r[...]
    h = _dot(h, w0x_r[...]) + b0x_r[...]
    h_r[...] = h
    hn = _ln_relu(h, g_r[...], bt_r[...])
    _split(u_r, _dot(hn, wma_r[...]) + bm_r[...])
    _split(a_r, _dot(hn, wea_r[...]) + bea_r[...])
    _split(b_r, _dot(hn, web_r[...]))


def _tc_node0(x, wne, bne, w0x, b0x, g, bt, wma, bm, wea, bea, web):
    tshp = jax.ShapeDtypeStruct((2, _N, _HH), _F32)
    return pl.pallas_call(
        _node0_body,
        grid=(_NG,),
        in_specs=[_row_spec_n, _w_spec, _b_spec, _w_spec, _b_spec, _b_spec,
                  _b_spec, _w_spec, _b_spec, _w_spec, _b_spec, _w_spec],
        out_specs=[_row_spec_n, _half_spec_n, _half_spec_n, _half_spec_n],
        out_shape=[jax.ShapeDtypeStruct((_N, _H), _F32), tshp, tshp, tshp],
    )(x, wne, bne, w0x, b0x, g, bt, wma, bm, wea, bea, web)


def _apply_agg(h_prev, aggp):
    agg = jnp.concatenate([aggp[0, :, :_HH], aggp[1, :, :_HH]], axis=-1)
    cnt = aggp[0, :, _HH:_HH + 1]
    return h_prev + agg / jnp.maximum(cnt, 1.0)


def _node_mid_body(h_r, aggp_r, g_r, bt_r, wma_r, bm_r, wea_r, bea_r, web_r,
                   hn_out_r, u_r, a_r, b_r):
    h = _apply_agg(h_r[...], aggp_r[...])
    hn_out_r[...] = h
    hn = _ln_relu(h, g_r[...], bt_r[...])
    _split(u_r, _dot(hn, wma_r[...]) + bm_r[...])
    _split(a_r, _dot(hn, wea_r[...]) + bea_r[...])
    _split(b_r, _dot(hn, web_r[...]))


def _tc_node_mid(h, aggp, g, bt, wma, bm, wea, bea, web):
    tshp = jax.ShapeDtypeStruct((2, _N, _HH), _F32)
    return pl.pallas_call(
        _node_mid_body,
        grid=(_NG,),
        in_specs=[_row_spec_n, _aggp_spec, _b_spec, _b_spec, _w_spec, _b_spec,
                  _w_spec, _b_spec, _w_spec],
        out_specs=[_row_spec_n, _half_spec_n, _half_spec_n, _half_spec_n],
        out_shape=[jax.ShapeDtypeStruct((_N, _H), _F32), tshp, tshp, tshp],
    )(h, aggp, g, bt, wma, bm, wea, bea, web)


def _node_fin_body(h_r, aggp_r, cnt_r, g_r, bt_r, wdab_r, pq_r):
    aggp = aggp_r[...]
    agg = jnp.concatenate([aggp[0], aggp[1]], axis=-1)
    h = h_r[...] + agg / jnp.maximum(cnt_r[...], 1.0)
    hf = _ln_relu(h, g_r[...], bt_r[...])
    pq_r[...] = _dot(hf, wdab_r[...])


def _tc_node_fin(h, aggp, cnt, g, bt, wdab):
    return pl.pallas_call(
        _node_fin_body,
        grid=(_NG,),
        in_specs=[_row_spec_n,
                  pl.BlockSpec((2, _NT, _HH), lambda i: (0, i, 0)),
                  pl.BlockSpec((_NT, 1), lambda i: (i, 0)),
                  _b_spec, _b_spec,
                  pl.BlockSpec((_H, 2), lambda i: (0, 0))],
        out_specs=pl.BlockSpec((_NT, 2), lambda i: (i, 0)),
        out_shape=jax.ShapeDtypeStruct((_N, 2), _F32),
    )(h, aggp, cnt, g, bt, wdab)


def _edge0_body(ea_r, wee_r, bee_r, w0e_r, b0e_r, g_r, bt_r, wmb_r, wec_r,
                e_r, pm_r, pe_r):
    e = _dot(ea_r[...], wee_r[...]) + bee_r[...]
    e = _dot(e, w0e_r[...]) + b0e_r[...]
    e_r[...] = e
    gn = _ln_relu(e, g_r[...], bt_r[...])
    pm_r[...] = _dot(gn, wmb_r[...])
    pe_r[...] = _dot(gn, wec_r[...])


def _tc_edge0(ea, wee, bee, w0e, b0e, g, bt, wmb, wec):
    shp = jax.ShapeDtypeStruct((_E, _H), _F32)
    return pl.pallas_call(
        _edge0_body,
        grid=(_EG,),
        in_specs=[pl.BlockSpec((_ET, 16), lambda i: (i, 0)),
                  pl.BlockSpec((16, _H), lambda i: (0, 0)),
                  _b_spec, _w_spec, _b_spec, _b_spec, _b_spec, _w_spec, _w_spec],
        out_specs=[_row_spec_e] * 3,
        out_shape=[shp] * 3,
    )(ea, wee, bee, w0e, b0e, g, bt, wmb, wec)


def _edge_mid_body(e_r, g_r, bt_r, wmb_r, wec_r, pm_r, pe_r):
    gn = _ln_relu(e_r[...], g_r[...], bt_r[...])
    pm_r[...] = _dot(gn, wmb_r[...])
    pe_r[...] = _dot(gn, wec_r[...])


def _tc_edge_mid(e, g, bt, wmb, wec):
    shp = jax.ShapeDtypeStruct((_E, _H), _F32)
    return pl.pallas_call(
        _edge_mid_body,
        grid=(_EG,),
        in_specs=[_row_spec_e, _b_spec, _b_spec, _w_spec, _w_spec],
        out_specs=[_row_spec_e] * 2,
        out_shape=[shp] * 2,
    )(e, g, bt, wmb, wec)


def _edge_fin_body(e_r, g_r, bt_r, wdc_r, bd_r, r_r):
    ef = _ln_relu(e_r[...], g_r[...], bt_r[...])
    r_r[...] = _dot(ef, wdc_r[...]) + bd_r[...]


def _tc_edge_fin(e, g, bt, wdc, bd):
    return pl.pallas_call(
        _edge_fin_body,
        grid=(_EG,),
        in_specs=[_row_spec_e, _b_spec, _b_spec,
                  pl.BlockSpec((_H, 1), lambda i: (0, 0)),
                  pl.BlockSpec((1, 1), lambda i: (0, 0))],
        out_specs=pl.BlockSpec((_ET, 1), lambda i: (i, 0)),
        out_shape=jax.ShapeDtypeStruct((_E, 1), _F32),
    )(e, g, bt, wdc, bd)


_EPS = _E // 16
_C = 80
_NCHUNK = _EPS // _C
_RPT = _N // 16
_ZR = 25
_QH = _HH // 16

_sc_mesh = plsc.VectorSubcoreMesh(core_axis_name="c", subcore_axis_name="s")
_sc_params = pltpu.CompilerParams(use_tc_tiling_on_sc=False,
                                  needs_layout_passes=False)


_CH = _C * _HH
_NPAIR = _NCHUNK // 2


def _sc_e_body(ei_hbm, e_hbm, pe_hbm, a_hbm, b_hbm, e_out,
               sidx, didx, gsx, gdx, abuf, bbuf, ebuf, pebuf, obuf,
               sem_i, sem_l, sem_g, sem_w):
    c = lax.axis_index("c")
    s = lax.axis_index("s")
    ebase = s * _EPS
    chb = c * _HH
    coff = c * _N

    def _fire_idx(i):
        q = lax.rem(i, 4)
        eoff = ebase + i * _C
        pltpu.async_copy(ei_hbm.at[0, pl.ds(eoff, _C)], sidx.at[q], sem_i)
        pltpu.async_copy(ei_hbm.at[1, pl.ds(eoff, _C)], didx.at[q], sem_i)

    def _fire_streams(b, i):
        eoff = ebase + i * _C
        hs = pl.ds(chb, _HH)
        bs = pl.ds(b * _C, _C)
        pltpu.async_copy(e_hbm.at[pl.ds(eoff, _C), hs], ebuf.at[bs], sem_l)
        pltpu.async_copy(pe_hbm.at[pl.ds(eoff, _C), hs], pebuf.at[bs], sem_l)

    def _arm_gathers(b, i):
        q = lax.rem(i, 4)
        pltpu.make_async_copy(ei_hbm.at[0, pl.ds(0, _C)], sidx.at[0],
                              sem_i).wait()
        pltpu.make_async_copy(ei_hbm.at[0, pl.ds(0, _C)], didx.at[0],
                              sem_i).wait()

        def _shift(j, _):
            sl = pl.ds(j * 16, 16)
            gsx[b, sl] = sidx[q, sl] + coff
            gdx[b, sl] = didx[q, sl] + coff
            return 0

        lax.fori_loop(0, _C // 16, _shift, 0)
        pltpu.async_copy(a_hbm.at[gsx.at[b]], abuf.at[pl.ds(b * _C, _C)],
                         sem_g)
        pltpu.async_copy(b_hbm.at[gdx.at[b]], bbuf.at[pl.ds(b * _C, _C)],
                         sem_g)

    def _drain_streams(b):
        for buf in (ebuf, pebuf):
            pltpu.make_async_copy(e_hbm.at[pl.ds(0, _C), pl.ds(0, _HH)],
                                  buf.at[pl.ds(b * _C, _C)], sem_l).wait()

    def _drain_gathers(b):
        for buf in (abuf, bbuf):
            pltpu.make_async_copy(a_hbm.at[pl.ds(0, _C)],
                                  buf.at[pl.ds(b * _C, _C)], sem_g).wait()

    def _compute(b):
        def _row(j, _):
            jj = b * _C + j
            for k in range(_QH):
                msl = pl.ds(k * 16, 16)
                obuf[j, msl] = ebuf[jj, msl] + jnp.maximum(
                    abuf[jj, msl] + bbuf[jj, msl] + pebuf[jj, msl], 0.0)
            return 0

        lax.fori_loop(0, _C, _row, 0)

    def _write(b, i):
        eoff = ebase + i * _C
        pltpu.async_copy(obuf, e_out.at[pl.ds(eoff, _C), pl.ds(chb, _HH)],
                         sem_w)

    def _drain_w():
        pltpu.make_async_copy(obuf, e_out.at[pl.ds(0, _C), pl.ds(0, _HH)],
                              sem_w).wait()

    _fire_idx(0)
    _fire_idx(1)
    _fire_idx(2)
    _fire_streams(0, 0)
    _fire_streams(1, 1)
    _arm_gathers(0, 0)

    def _pair(t, _):
        i0 = 2 * t
        _arm_gathers(1, i0 + 1)

        @pl.when(i0 + 3 < _NCHUNK)
        def _():
            _fire_idx(i0 + 3)

        _drain_streams(0)
        _drain_gathers(0)

        @pl.when(t > 0)
        def _():
            _drain_w()

        _compute(0)
        _write(0, i0)

        @pl.when(i0 + 2 < _NCHUNK)
        def _():
            _fire_streams(0, i0 + 2)
            _arm_gathers(0, i0 + 2)

        @pl.when(i0 + 4 < _NCHUNK)
        def _():
            _fire_idx(i0 + 4)

        _drain_streams(1)
        _drain_gathers(1)
        _drain_w()
        _compute(1)
        _write(1, i0 + 1)

        @pl.when(i0 + 3 < _NCHUNK)
        def _():
            _fire_streams(1, i0 + 3)

        return 0

    lax.fori_loop(0, _NPAIR, _pair, 0)
    _drain_w()


_sc_e = functools.partial(
    pl.kernel,
    mesh=_sc_mesh,
    compiler_params=_sc_params,
    out_type=jax.ShapeDtypeStruct((_E, _H), _F32),
    scratch_types=[
        pltpu.VMEM((4, _C), jnp.int32),
        pltpu.VMEM((4, _C), jnp.int32),
        pltpu.VMEM((2, _C), jnp.int32),
        pltpu.VMEM((2, _C), jnp.int32),
        pltpu.VMEM((2 * _C, _HH), _F32),
        pltpu.VMEM((2 * _C, _HH), _F32),
        pltpu.VMEM((2 * _C, _HH), _F32),
        pltpu.VMEM((2 * _C, _HH), _F32),
        pltpu.VMEM((_C, _HH), _F32),
        pltpu.SemaphoreType.DMA,
        pltpu.SemaphoreType.DMA,
        pltpu.SemaphoreType.DMA,
        pltpu.SemaphoreType.DMA,
    ],
)(_sc_e_body)


def _make_sc_m(aggw, with_count):
    def _sc_m_body(ei_hbm, pm_hbm, u_hbm, edep_hbm, agg_out,
                   sidx, didx, gsx, dcp, ubuf, pmbuf, mbuf, zrow, acc,
                   sem_i, sem_l, sem_g, sem_s):
        del edep_hbm
        c = lax.axis_index("c")
        s = lax.axis_index("s")
        ebase = s * _EPS
        chb = c * _HH
        coff = c * _N

        zv = jnp.zeros((16,), _F32)

        def _zfill(j, _):
            for k in range(aggw // 16):
                zrow[j, pl.ds(k * 16, 16)] = zv
            return 0

        lax.fori_loop(0, _ZR, _zfill, 0)

        def _zcopy(t, _):
            pltpu.sync_copy(zrow, acc.at[pl.ds(s * _RPT + t * _ZR, _ZR)])
            return 0

        lax.fori_loop(0, _RPT // _ZR, _zcopy, 0)
        plsc.subcore_barrier()

        if with_count:
            padv = jnp.where(lax.iota(jnp.int32, 16) == 0,
                             jnp.full((16,), 1.0, _F32),
                             jnp.zeros((16,), _F32))

            def _padfill(j, _):
                mbuf[j, pl.ds(_HH, 16)] = padv
                return 0

            lax.fori_loop(0, 2 * _C, _padfill, 0)

        def _fire_idx(i):
            q = lax.rem(i, 4)
            eoff = ebase + i * _C
            pltpu.async_copy(ei_hbm.at[0, pl.ds(eoff, _C)], sidx.at[q], sem_i)
            pltpu.async_copy(ei_hbm.at[1, pl.ds(eoff, _C)], didx.at[q], sem_i)

        def _fire_streams(b, i):
            eoff = ebase + i * _C
            pltpu.async_copy(pm_hbm.at[pl.ds(eoff, _C), pl.ds(chb, _HH)],
                             pmbuf.at[pl.ds(b * _C, _C)], sem_l)

        def _arm_gathers(b, i):
            q = lax.rem(i, 4)
            pltpu.make_async_copy(ei_hbm.at[0, pl.ds(0, _C)], sidx.at[0],
                                  sem_i).wait()
            pltpu.make_async_copy(ei_hbm.at[0, pl.ds(0, _C)], didx.at[0],
                                  sem_i).wait()

            def _shift(j, _):
                sl = pl.ds(j * 16, 16)
                gsx[b, sl] = sidx[q, sl] + coff
                dcp[q, sl] = didx[q, sl]
                return 0

            lax.fori_loop(0, _C // 16, _shift, 0)
            pltpu.async_copy(u_hbm.at[gsx.at[b]], ubuf.at[pl.ds(b * _C, _C)],
                             sem_g)

        def _drain_streams(b):
            pltpu.make_async_copy(pm_hbm.at[pl.ds(0, _C), pl.ds(0, _HH)],
                                  pmbuf.at[pl.ds(b * _C, _C)], sem_l).wait()

        def _drain_gathers(b):
            pltpu.make_async_copy(u_hbm.at[pl.ds(0, _C)],
                                  ubuf.at[pl.ds(b * _C, _C)], sem_g).wait()

        def _compute(b):
            def _row(j, _):
                jj = b * _C + j
                for k in range(_QH):
                    msl = pl.ds(k * 16, 16)
                    mbuf[jj, msl] = jnp.maximum(ubuf[jj, msl] + pmbuf[jj, msl],
                                                0.0)
                return 0

            lax.fori_loop(0, _C, _row, 0)

        def _write(b, i):
            q = lax.rem(i, 4)
            pltpu.async_copy(mbuf.at[pl.ds(b * _C, _C)], acc.at[dcp.at[q]],
                             sem_s, add=True)

        def _drain_s(b):
            pltpu.make_async_copy(mbuf.at[pl.ds(b * _C, _C)],
                                  acc.at[pl.ds(0, _C)], sem_s).wait()

        _fire_idx(0)
        _fire_idx(1)
        _fire_idx(2)
        _fire_streams(0, 0)
        _fire_streams(1, 1)
        _arm_gathers(0, 0)

        def _pair(t, _):
            i0 = 2 * t
            _arm_gathers(1, i0 + 1)

            @pl.when(i0 + 3 < _NCHUNK)
            def _():
                _fire_idx(i0 + 3)

            _drain_streams(0)
            _drain_gathers(0)

            @pl.when(t > 0)
            def _():
                _drain_s(0)

            _compute(0)
            _write(0, i0)

            @pl.when(i0 + 2 < _NCHUNK)
            def _():
                _fire_streams(0, i0 + 2)
                _arm_gathers(0, i0 + 2)

            @pl.when(i0 + 4 < _NCHUNK)
            def _():
                _fire_idx(i0 + 4)

            _drain_streams(1)
            _drain_gathers(1)

            @pl.when(t > 0)
            def _():
                _drain_s(1)

            _compute(1)
            _write(1, i0 + 1)

            @pl.when(i0 + 3 < _NCHUNK)
            def _():
                _fire_streams(1, i0 + 3)

            return 0

        lax.fori_loop(0, _NPAIR, _pair, 0)
        _drain_s(0)
        _drain_s(1)
        plsc.subcore_barrier()

        def _flush(t, _):
            r0 = s * _RPT + t * _ZR
            pltpu.sync_copy(acc.at[pl.ds(r0, _ZR)], agg_out.at[c, pl.ds(r0, _ZR)])
            return 0

        lax.fori_loop(0, _RPT // _ZR, _flush, 0)


    _sc_m_kernel = functools.partial(
        pl.kernel,
        mesh=_sc_mesh,
        compiler_params=_sc_params,
        out_type=jax.ShapeDtypeStruct((2, _N, aggw), _F32),
        scratch_types=[
            pltpu.VMEM((4, _C), jnp.int32),
            pltpu.VMEM((4, _C), jnp.int32),
            pltpu.VMEM((2, _C), jnp.int32),
            pltpu.VMEM((4, _C), jnp.int32),
            pltpu.VMEM((2 * _C, _HH), _F32),
            pltpu.VMEM((2 * _C, _HH), _F32),
            pltpu.VMEM((2 * _C, aggw), _F32),
            pltpu.VMEM((_ZR, aggw), _F32),
            pltpu.VMEM_SHARED((_N, aggw), _F32),
            pltpu.SemaphoreType.DMA,
            pltpu.SemaphoreType.DMA,
            pltpu.SemaphoreType.DMA,
            pltpu.SemaphoreType.DMA,
        ],
    )(_sc_m_body)
    return _sc_m_kernel


_sc_m = _make_sc_m(_AGGW, True)
_sc_m2 = _make_sc_m(_HH, False)

_EPW = _E // 32
_C2 = 2000


def _sc_dec_body(ei_hbm, r_hbm, p_hbm, q_hbm, z_out,
                 ptab, qtab, sbuf, dbuf, rbuf, zbuf):
    c = lax.axis_index("c")
    s = lax.axis_index("s")
    w = c * 16 + s
    base = w * _EPW
    pltpu.sync_copy(p_hbm, ptab)
    pltpu.sync_copy(q_hbm, qtab)

    def _chunk(i, _):
        off = base + i * _C2
        pltpu.sync_copy(ei_hbm.at[0, pl.ds(off, _C2)], sbuf)
        pltpu.sync_copy(ei_hbm.at[1, pl.ds(off, _C2)], dbuf)
        pltpu.sync_copy(r_hbm.at[pl.ds(off, _C2)], rbuf)

        def _vec(t, _):
            sl = pl.ds(t * 16, 16)
            pg = plsc.load_gather(ptab, [sbuf[sl]])
            qg = plsc.load_gather(qtab, [dbuf[sl]])
            zbuf[sl] = jnp.maximum(pg + qg + rbuf[sl], 0.0)
            return 0

        lax.fori_loop(0, _C2 // 16, _vec, 0)
        pltpu.sync_copy(zbuf, z_out.at[pl.ds(off, _C2)])
        return 0

    lax.fori_loop(0, _EPW // _C2, _chunk, 0)


_sc_dec = functools.partial(
    pl.kernel,
    mesh=_sc_mesh,
    compiler_params=_sc_params,
    out_type=jax.ShapeDtypeStruct((_E,), _F32),
    scratch_types=[
        pltpu.VMEM((_N,), _F32),
        pltpu.VMEM((_N,), _F32),
        pltpu.VMEM((_C2,), jnp.int32),
        pltpu.VMEM((_C2,), jnp.int32),
        pltpu.VMEM((_C2,), _F32),
        pltpu.VMEM((_C2,), _F32),
    ],
)(_sc_dec_body)


def kernel(x, edge_index, edge_attr, W_ne, b_ne, W_ee, b_ee, W0x, b0x, W0e,
           b0e, gamma, beta, Wm, bm, We, be, W_dec, b_dec, scale):
    r1 = lambda v: v.reshape(1, _H)
    tab = lambda v: v.reshape(2 * _N, _HH)
    g = [r1(gamma[0]), r1(gamma[1])]
    bt = [r1(beta[0]), r1(beta[1])]
    wma = [Wm[0, :_H], Wm[1, :_H]]
    wmb = [Wm[0, _H:], Wm[1, _H:]]
    wea = [We[0, :_H], We[1, :_H]]
    web = [We[0, _H:2 * _H], We[1, _H:2 * _H]]
    wec = [We[0, 2 * _H:], We[1, 2 * _H:]]
    bmr = [r1(bm[0]), r1(bm[1])]
    ber = [r1(be[0]), r1(be[1])]
    wdab = W_dec[:2 * _H].reshape(2, _H).T
    wdc = W_dec[2 * _H:]
    bd = b_dec.reshape(1, 1)

    h0, u0, a0, b0 = _tc_node0(x, W_ne, r1(b_ne), W0x, r1(b0x), g[0], bt[0],
                               wma[0], bmr[0], wea[0], ber[0], web[0])
    e0, pm0, pe0 = _tc_edge0(edge_attr, W_ee, r1(b_ee), W0e, r1(b0e), g[0],
                             bt[0], wmb[0], wec[0])
    e1 = _sc_e(edge_index, e0, pe0, tab(a0), tab(b0))
    aggp0 = _sc_m(edge_index, pm0, tab(u0), e1)
    pm1, pe1 = _tc_edge_mid(e1, g[1], bt[1], wmb[1], wec[1])
    h1, u1, a1, b1 = _tc_node_mid(h0, aggp0, g[1], bt[1], wma[1], bmr[1],
                                  wea[1], ber[1], web[1])
    e2 = _sc_e(edge_index, e1, pe1, tab(a1), tab(b1))
    aggp1 = _sc_m2(edge_index, pm1, tab(u1), e2)
    r = _tc_edge_fin(e2, g[0], bt[0], wdc, bd)
    cnt0 = aggp0[0, :, _HH:_HH + 1]
    pq = _tc_node_fin(h1, aggp1, cnt0, g[0], bt[0], wdab)
    z = _sc_dec(edge_index, r.reshape(_E), pq[:, 0], pq[:, 1])
    return (z * scale).reshape(_E, 1)
